# baseline (speedup 1.0000x reference)
import functools

import jax
import jax.numpy as jnp
from jax import lax
from jax.experimental import pallas as pl
from jax.experimental.pallas import tpu as pltpu
from jax.experimental.pallas import tpu_sc as plsc

N_ATOMS = 100000
N_PAIRS = 3200000
E_TYPES = 3
N_R = 8192
N_RHO = 4096
R_MAX = 6.0
INV_DR = (N_R - 1) / R_MAX
RMAX_C = R_MAX * (1.0 - 1e-07)
RHO_CLIP_HI = N_RHO - 1 - 1e-04

NC = 2
NS = 16
NW = NC * NS
L = 16

NA_PAD = 100352
AT_W = NA_PAD // NW
AT_ROWS = AT_W // L
AT_SC = NA_PAD // NS

PAIRS_W = N_PAIRS // NW
CHUNK = 800
VECS = CHUNK // L
NCHUNK = PAIRS_W // CHUNK

DENS_N = E_TYPES * N_R
PAIR_N = E_TYPES * E_TYPES * N_R
OUT_LEN = N_ATOMS + N_PAIRS
ECHUNKS = N_ATOMS // CHUNK

f32 = jnp.float32
i32 = jnp.int32


def _iota16():
    return lax.iota(i32, L)


def _rbin(rr):
    rc = jnp.minimum(jnp.maximum(rr, 0.0), RMAX_C)
    idxf = rc * INV_DR
    idx = idxf.astype(i32)
    frac = idxf - idx.astype(f32)
    return idx, frac


def _vloop(n, body):
    def wrap(i, carry):
        body(i)
        return carry
    lax.fori_loop(0, n, wrap, 0)


def _zero_fill(ref, nwords):
    zeros16 = jnp.zeros((L,), f32)

    def zf(i):
        ref[pl.ds(i * L, L)] = zeros16

    _vloop(nwords // L, zf)


def _pair_pass1_body(r1, s1, d1, tpad, dflat, pflat,
                     rho_out, pe_out, dbi_out, dbj_out,
                     type_s, dens_ts, pair_ts, acc,
                     r_b, s_b, d_b, ti_b, tj_b, i0_b, i1_b, v0_b, v1_b):
    c = lax.axis_index("c")
    s = lax.axis_index("s")
    w = c * NS + s

    sl = pl.ds(s * AT_SC, AT_SC)
    pltpu.sync_copy(tpad.at[sl], type_s.at[sl])
    dsl = pl.ds(s * (DENS_N // NS), DENS_N // NS)
    pltpu.sync_copy(dflat.at[dsl], dens_ts.at[dsl])
    psl = pl.ds(s * (PAIR_N // NS), PAIR_N // NS)
    pltpu.sync_copy(pflat.at[psl], pair_ts.at[psl])
    _zero_fill(acc, NA_PAD)
    plsc.subcore_barrier()

    def chunk1(ci):
        base = w * PAIRS_W + ci * CHUNK
        pltpu.sync_copy(r1.at[pl.ds(base, CHUNK)], r_b)
        pltpu.sync_copy(s1.at[pl.ds(base, CHUNK)], s_b)
        pltpu.sync_copy(d1.at[pl.ds(base, CHUNK)], d_b)
        pltpu.sync_copy(type_s.at[s_b], ti_b)
        pltpu.sync_copy(type_s.at[d_b], tj_b)

        def idxrow(i):
            rsl = pl.ds(i * L, L)
            idx, _ = _rbin(r_b[rsl])
            dbi = ti_b[rsl] * N_R
            dbj = tj_b[rsl] * N_R
            fi = dbi * E_TYPES + dbj + idx
            i0_b[rsl] = fi
            i1_b[rsl] = fi + 1
            ti_b[rsl] = dbi
            tj_b[rsl] = dbj

        _vloop(VECS, idxrow)
        pltpu.sync_copy(pair_ts.at[i0_b], v0_b)
        pltpu.sync_copy(pair_ts.at[i1_b], v1_b)
        pltpu.sync_copy(ti_b, dbi_out.at[pl.ds(base, CHUNK)])
        pltpu.sync_copy(tj_b, dbj_out.at[pl.ds(base, CHUNK)])

        def accrow(i):
            rsl = pl.ds(i * L, L)
            _, frac = _rbin(r_b[rsl])
            v0 = v0_b[rsl]
            phi = v0 + frac * (v1_b[rsl] - v0)
            plsc.addupdate_scatter(acc, [s_b[rsl]], 0.5 * phi)

        _vloop(VECS, accrow)

    _vloop(NCHUNK, chunk1)
    pltpu.sync_copy(acc, pe_out.at[pl.ds(w * NA_PAD, NA_PAD)])
    _zero_fill(acc, NA_PAD)

    def chunk2(ci):
        base = w * PAIRS_W + ci * CHUNK
        pltpu.sync_copy(r1.at[pl.ds(base, CHUNK)], r_b)
        pltpu.sync_copy(s1.at[pl.ds(base, CHUNK)], s_b)
        pltpu.sync_copy(dbj_out.at[pl.ds(base, CHUNK)], tj_b)

        def idxrow(i):
            rsl = pl.ds(i * L, L)
            idx, _ = _rbin(r_b[rsl])
            fi = tj_b[rsl] + idx
            i0_b[rsl] = fi
            i1_b[rsl] = fi + 1

        _vloop(VECS, idxrow)
        pltpu.sync_copy(dens_ts.at[i0_b], v0_b)
        pltpu.sync_copy(dens_ts.at[i1_b], v1_b)

        def accrow(i):
            rsl = pl.ds(i * L, L)
            _, frac = _rbin(r_b[rsl])
            v0 = v0_b[rsl]
            dens = v0 + frac * (v1_b[rsl] - v0)
            plsc.addupdate_scatter(acc, [s_b[rsl]], dens)

        _vloop(VECS, accrow)

    _vloop(NCHUNK, chunk2)
    pltpu.sync_copy(acc, rho_out.at[pl.ds(w * NA_PAD, NA_PAD)])


def _atom_pass_body(rho_part, pe_part, tpad, eflat, epflat, rmin16, idr16,
                    en_out, fp_out,
                    embed_t, embedp_t, rmin_t, idr_t,
                    rho_b, pe_b, tmp_b, tb, en_b, fp_b):
    c = lax.axis_index("c")
    s = lax.axis_index("s")
    w = c * NS + s
    base = pl.ds(w * AT_W, AT_W)

    pltpu.sync_copy(eflat, embed_t)
    pltpu.sync_copy(epflat, embedp_t)
    pltpu.sync_copy(rmin16, rmin_t)
    pltpu.sync_copy(idr16, idr_t)
    pltpu.sync_copy(tpad.at[base], tb)

    pltpu.sync_copy(rho_part.at[pl.ds(w * AT_W, AT_W)], rho_b)
    pltpu.sync_copy(pe_part.at[pl.ds(w * AT_W, AT_W)], pe_b)

    def red(p):
        pltpu.sync_copy(rho_part.at[pl.ds(p * NA_PAD + w * AT_W, AT_W)],
                        tmp_b)

        def addrow_r(j):
            sl = pl.ds(j * L, L)
            rho_b[sl] = rho_b[sl] + tmp_b[sl]

        _vloop(AT_ROWS, addrow_r)
        pltpu.sync_copy(pe_part.at[pl.ds(p * NA_PAD + w * AT_W, AT_W)],
                        tmp_b)

        def addrow_p(j):
            sl = pl.ds(j * L, L)
            pe_b[sl] = pe_b[sl] + tmp_b[sl]

        _vloop(AT_ROWS, addrow_p)

    def redwrap(p, carry):
        red(p + 1)
        return carry

    lax.fori_loop(0, NW - 1, redwrap, 0)

    def row(j):
        sl = pl.ds(j * L, L)
        t = tb[sl]
        rho = rho_b[sl]
        rm = plsc.load_gather(rmin_t, [t])
        iv = plsc.load_gather(idr_t, [t])
        idxf = jnp.minimum(jnp.maximum((rho - rm) * iv, 0.0), RHO_CLIP_HI)
        idx = idxf.astype(i32)
        frac = idxf - idx.astype(f32)
        eb = t * N_RHO + idx
        F0 = plsc.load_gather(embed_t, [eb])
        F1 = plsc.load_gather(embed_t, [eb + 1])
        G0 = plsc.load_gather(embedp_t, [eb])
        G1 = plsc.load_gather(embedp_t, [eb + 1])
        en_b[sl] = F0 + frac * (F1 - F0) + pe_b[sl]
        fp_b[sl] = G0 + frac * (G1 - G0)

    _vloop(AT_ROWS, row)
    pltpu.sync_copy(en_b, en_out.at[base])
    pltpu.sync_copy(fp_b, fp_out.at[base])


def _pair_pass2_body(r1, s1, d1, dbi, dbj, fp_pad, dpflat, ppflat, en_pad,
                     out1,
                     fp_s, densp_t, pairp_t,
                     r_b, s_b, d_b, bi_b, bj_b, fs_b, fd_b, f_b):
    c = lax.axis_index("c")
    s = lax.axis_index("s")
    w = c * NS + s

    sl = pl.ds(s * AT_SC, AT_SC)
    pltpu.sync_copy(fp_pad.at[sl], fp_s.at[sl])
    pltpu.sync_copy(dpflat, densp_t)
    pltpu.sync_copy(ppflat, pairp_t)

    nch = ECHUNKS // NW + jnp.where(w < ECHUNKS % NW, 1, 0)

    def ecopy(k):
        ebase = (w + k * NW) * CHUNK
        pltpu.sync_copy(en_pad.at[pl.ds(ebase, CHUNK)], f_b)
        pltpu.sync_copy(f_b, out1.at[pl.ds(ebase, CHUNK)])

    _vloop(nch, ecopy)
    plsc.subcore_barrier()

    def chunk(ci):
        base = w * PAIRS_W + ci * CHUNK
        pltpu.sync_copy(r1.at[pl.ds(base, CHUNK)], r_b)
        pltpu.sync_copy(s1.at[pl.ds(base, CHUNK)], s_b)
        pltpu.sync_copy(d1.at[pl.ds(base, CHUNK)], d_b)
        pltpu.sync_copy(dbi.at[pl.ds(base, CHUNK)], bi_b)
        pltpu.sync_copy(dbj.at[pl.ds(base, CHUNK)], bj_b)
        pltpu.sync_copy(fp_s.at[s_b], fs_b)
        pltpu.sync_copy(fp_s.at[d_b], fd_b)

        def row(i):
            rsl = pl.ds(i * L, L)
            idx, frac = _rbin(r_b[rsl])
            bi = bi_b[rsl]
            bj = bj_b[rsl]
            pb = bi * E_TYPES + bj + idx
            p0 = plsc.load_gather(pairp_t, [pb])
            p1 = plsc.load_gather(pairp_t, [pb + 1])
            phip = p0 + frac * (p1 - p0)
            j0 = plsc.load_gather(densp_t, [bj + idx])
            j1 = plsc.load_gather(densp_t, [bj + idx + 1])
            rhop_j = j0 + frac * (j1 - j0)
            q0 = plsc.load_gather(densp_t, [bi + idx])
            q1 = plsc.load_gather(densp_t, [bi + idx + 1])
            rhop_i = q0 + frac * (q1 - q0)
            f_b[rsl] = phip + fs_b[rsl] * rhop_j + fd_b[rsl] * rhop_i

        _vloop(VECS, row)
        pltpu.sync_copy(f_b, out1.at[pl.ds(N_ATOMS + base, CHUNK)])

    _vloop(NCHUNK, chunk)


@functools.cache
def _build(interpret=False):
    def mesh():
        return plsc.VectorSubcoreMesh(core_axis_name="c",
                                      subcore_axis_name="s")

    params = pltpu.CompilerParams(needs_layout_passes=False)

    pass1 = pl.kernel(
        _pair_pass1_body,
        out_type=(
            jax.ShapeDtypeStruct((NW * NA_PAD,), f32),
            jax.ShapeDtypeStruct((NW * NA_PAD,), f32),
            jax.ShapeDtypeStruct((N_PAIRS,), i32),
            jax.ShapeDtypeStruct((N_PAIRS,), i32),
        ),
        mesh=mesh(),
        interpret=interpret,
        compiler_params=params,
        scratch_types=[
            pltpu.VMEM_SHARED((NA_PAD,), i32),
            pltpu.VMEM_SHARED((DENS_N,), f32),
            pltpu.VMEM_SHARED((PAIR_N,), f32),
            pltpu.VMEM((NA_PAD,), f32),
            pltpu.VMEM((CHUNK,), f32),
            pltpu.VMEM((CHUNK,), i32),
            pltpu.VMEM((CHUNK,), i32),
            pltpu.VMEM((CHUNK,), i32),
            pltpu.VMEM((CHUNK,), i32),
            pltpu.VMEM((CHUNK,), i32),
            pltpu.VMEM((CHUNK,), i32),
            pltpu.VMEM((CHUNK,), f32),
            pltpu.VMEM((CHUNK,), f32),
        ],
    )

    pass_b = pl.kernel(
        _atom_pass_body,
        out_type=(
            jax.ShapeDtypeStruct((NA_PAD,), f32),
            jax.ShapeDtypeStruct((NA_PAD,), f32),
        ),
        mesh=mesh(),
        interpret=interpret,
        compiler_params=params,
        scratch_types=[
            pltpu.VMEM((E_TYPES * N_RHO,), f32),
            pltpu.VMEM((E_TYPES * N_RHO,), f32),
            pltpu.VMEM((L,), f32),
            pltpu.VMEM((L,), f32),
            pltpu.VMEM((AT_W,), f32),
            pltpu.VMEM((AT_W,), f32),
            pltpu.VMEM((AT_W,), f32),
            pltpu.VMEM((AT_W,), i32),
            pltpu.VMEM((AT_W,), f32),
            pltpu.VMEM((AT_W,), f32),
        ],
    )

    pass2 = pl.kernel(
        _pair_pass2_body,
        out_type=jax.ShapeDtypeStruct((OUT_LEN,), f32),
        mesh=mesh(),
        interpret=interpret,
        compiler_params=params,
        scratch_types=[
            pltpu.VMEM_SHARED((NA_PAD,), f32),
            pltpu.VMEM((DENS_N,), f32),
            pltpu.VMEM((PAIR_N,), f32),
            pltpu.VMEM((CHUNK,), f32),
            pltpu.VMEM((CHUNK,), i32),
            pltpu.VMEM((CHUNK,), i32),
            pltpu.VMEM((CHUNK,), i32),
            pltpu.VMEM((CHUNK,), i32),
            pltpu.VMEM((CHUNK,), f32),
            pltpu.VMEM((CHUNK,), f32),
            pltpu.VMEM((CHUNK,), f32),
        ],
    )
    return pass1, pass_b, pass2


def _run(r, edge_index, atom_type_indices, density_table,
         density_deriv_table, pair_table, pair_deriv_table,
         embed_table, embed_deriv_table, embed_rho_min, embed_inv_drho,
         interpret=False):
    pass1, pass_b, pass2 = _build(interpret)
    src1 = edge_index[0]
    dst1 = edge_index[1]
    tpad = jnp.pad(atom_type_indices, (0, NA_PAD - N_ATOMS))
    rmin16 = jnp.pad(embed_rho_min, (0, L - E_TYPES))
    idr16 = jnp.pad(embed_inv_drho, (0, L - E_TYPES))
    rho_part, pe_part, dbi, dbj = pass1(
        r, src1, dst1, tpad,
        density_table.reshape(-1), pair_table.reshape(-1))
    en_pad, fp_pad = pass_b(
        rho_part, pe_part, tpad,
        embed_table.reshape(-1), embed_deriv_table.reshape(-1),
        rmin16, idr16)
    return pass2(
        r, src1, dst1, dbi, dbj, fp_pad,
        density_deriv_table.reshape(-1), pair_deriv_table.reshape(-1),
        en_pad)


def kernel(r, edge_index, atom_type_indices, density_table,
           density_deriv_table, pair_table, pair_deriv_table,
           embed_table, embed_deriv_table, embed_rho_min, embed_inv_drho):
    return _run(r, edge_index, atom_type_indices, density_table,
                density_deriv_table, pair_table, pair_deriv_table,
                embed_table, embed_deriv_table, embed_rho_min,
                embed_inv_drho)


# speedup vs baseline: 1.7696x; 1.7696x over previous
import functools

import jax
import jax.numpy as jnp
from jax import lax
from jax.experimental import pallas as pl
from jax.experimental.pallas import tpu as pltpu
from jax.experimental.pallas import tpu_sc as plsc

N_ATOMS = 100000
N_PAIRS = 3200000
E_TYPES = 3
N_R = 8192
N_RHO = 4096
R_MAX = 6.0
INV_DR = (N_R - 1) / R_MAX
RMAX_C = R_MAX * (1.0 - 1e-07)
RHO_CLIP_HI = N_RHO - 1 - 1e-04

NC = 2
NS = 16
NW = NC * NS
L = 16

NA_PAD = 100352
AT_W = NA_PAD // NW
AT_ROWS = AT_W // L
AT_SC = NA_PAD // NS

PAIRS_T = N_PAIRS // NS
CHUNK_A = 1600
VECS_A = CHUNK_A // L
NCHUNK_A = PAIRS_T // CHUNK_A

PAIRS_W = N_PAIRS // NW
CHUNK_C = 2000
VECS_C = CHUNK_C // L
NCHUNK_C = PAIRS_W // CHUNK_C

DENS_N = E_TYPES * N_R
PAIR_N = E_TYPES * E_TYPES * N_R
OUT_LEN = N_ATOMS + N_PAIRS
ECHUNKS = N_ATOMS // CHUNK_C

f32 = jnp.float32
i32 = jnp.int32


def _rbin(rr):
    rc = jnp.minimum(jnp.maximum(rr, 0.0), RMAX_C)
    idxf = rc * INV_DR
    idx = idxf.astype(i32)
    frac = idxf - idx.astype(f32)
    return idx, frac


def _vloop(n, body):
    def wrap(i, carry):
        body(i)
        return carry
    lax.fori_loop(0, n, wrap, 0)


def _zero_fill(ref, nwords):
    zeros16 = jnp.zeros((L,), f32)

    def zf(i):
        ref[pl.ds(i * L, L)] = zeros16

    _vloop(nwords // L, zf)


def _pair_pass1_body(r1, s1, d1, tpad, dflat, pflat,
                     rho_out, pe_out, dbi_out, dbj_out,
                     type_s, dens_ts, pair_ts, acc,
                     r_b, s_b, d_b, ti_b, tj_b, i0_b, i1_b, v0_b, v1_b,
                     sem_in, sem_st, sem_out):
    c = lax.axis_index("c")
    s = lax.axis_index("s")

    sl = pl.ds(s * AT_SC, AT_SC)
    pltpu.sync_copy(tpad.at[sl], type_s.at[sl])
    dsl = pl.ds(s * (DENS_N // NS), DENS_N // NS)
    pltpu.sync_copy(dflat.at[dsl], dens_ts.at[dsl])
    psl = pl.ds(s * (PAIR_N // NS), PAIR_N // NS)
    pltpu.sync_copy(pflat.at[psl], pair_ts.at[psl])
    _zero_fill(acc, NA_PAD)
    plsc.subcore_barrier()

    @pl.when(c == 0)
    def _():
        def chunk(ci):
            base = s * PAIRS_T + ci * CHUNK_A
            cp_r = pltpu.async_copy(r1.at[pl.ds(base, CHUNK_A)], r_b,
                                    sem_in)
            cp_s = pltpu.async_copy(s1.at[pl.ds(base, CHUNK_A)], s_b,
                                    sem_in)
            cp_d = pltpu.async_copy(d1.at[pl.ds(base, CHUNK_A)], d_b,
                                    sem_in)
            cp_r.wait()
            cp_s.wait()
            cp_d.wait()
            g_i = pltpu.async_copy(type_s.at[s_b], ti_b, sem_st)
            g_j = pltpu.async_copy(type_s.at[d_b], tj_b, sem_st)
            g_i.wait()
            g_j.wait()

            def idxrow(i):
                rsl = pl.ds(i * L, L)
                idx, _ = _rbin(r_b[rsl])
                dbi = ti_b[rsl] * N_R
                dbj = tj_b[rsl] * N_R
                fi = dbi * E_TYPES + dbj + idx
                i0_b[rsl] = fi
                i1_b[rsl] = fi + 1
                ti_b[rsl] = dbi
                tj_b[rsl] = dbj

            _vloop(VECS_A, idxrow)
            g_0 = pltpu.async_copy(pair_ts.at[i0_b], v0_b, sem_st)
            g_1 = pltpu.async_copy(pair_ts.at[i1_b], v1_b, sem_st)
            o_i = pltpu.async_copy(ti_b, dbi_out.at[pl.ds(base, CHUNK_A)],
                                   sem_out)
            o_j = pltpu.async_copy(tj_b, dbj_out.at[pl.ds(base, CHUNK_A)],
                                   sem_out)
            g_0.wait()
            g_1.wait()

            def accrow(i):
                rsl = pl.ds(i * L, L)
                _, frac = _rbin(r_b[rsl])
                v0 = v0_b[rsl]
                phi = v0 + frac * (v1_b[rsl] - v0)
                plsc.addupdate_scatter(acc, [s_b[rsl]], 0.5 * phi)

            _vloop(VECS_A, accrow)
            o_i.wait()
            o_j.wait()

        _vloop(NCHUNK_A, chunk)
        pltpu.sync_copy(acc, pe_out.at[pl.ds(s * NA_PAD, NA_PAD)])

    @pl.when(c == 1)
    def _():
        def chunk(ci):
            base = s * PAIRS_T + ci * CHUNK_A
            cp_r = pltpu.async_copy(r1.at[pl.ds(base, CHUNK_A)], r_b,
                                    sem_in)
            cp_s = pltpu.async_copy(s1.at[pl.ds(base, CHUNK_A)], s_b,
                                    sem_in)
            cp_d = pltpu.async_copy(d1.at[pl.ds(base, CHUNK_A)], d_b,
                                    sem_in)
            cp_r.wait()
            cp_s.wait()
            cp_d.wait()
            g_j = pltpu.async_copy(type_s.at[d_b], tj_b, sem_st)
            g_j.wait()

            def idxrow(i):
                rsl = pl.ds(i * L, L)
                idx, _ = _rbin(r_b[rsl])
                fi = tj_b[rsl] * N_R + idx
                i0_b[rsl] = fi
                i1_b[rsl] = fi + 1

            _vloop(VECS_A, idxrow)
            g_0 = pltpu.async_copy(dens_ts.at[i0_b], v0_b, sem_st)
            g_1 = pltpu.async_copy(dens_ts.at[i1_b], v1_b, sem_st)
            g_0.wait()
            g_1.wait()

            def accrow(i):
                rsl = pl.ds(i * L, L)
                _, frac = _rbin(r_b[rsl])
                v0 = v0_b[rsl]
                dens = v0 + frac * (v1_b[rsl] - v0)
                plsc.addupdate_scatter(acc, [s_b[rsl]], dens)

            _vloop(VECS_A, accrow)

        _vloop(NCHUNK_A, chunk)
        pltpu.sync_copy(acc, rho_out.at[pl.ds(s * NA_PAD, NA_PAD)])


def _atom_pass_body(rho_part, pe_part, tpad, eflat, epflat, rmin16, idr16,
                    en_out, fp_out,
                    embed_t, embedp_t, rmin_t, idr_t,
                    rho_b, pe_b, tmp_b, tmp2_b, tb, en_b, fp_b, sem_r):
    c = lax.axis_index("c")
    s = lax.axis_index("s")
    w = c * NS + s
    base = pl.ds(w * AT_W, AT_W)

    pltpu.sync_copy(eflat, embed_t)
    pltpu.sync_copy(epflat, embedp_t)
    pltpu.sync_copy(rmin16, rmin_t)
    pltpu.sync_copy(idr16, idr_t)
    pltpu.sync_copy(tpad.at[base], tb)

    pltpu.sync_copy(rho_part.at[pl.ds(w * AT_W, AT_W)], rho_b)
    pltpu.sync_copy(pe_part.at[pl.ds(w * AT_W, AT_W)], pe_b)

    def red(p):
        cp_r = pltpu.async_copy(
            rho_part.at[pl.ds(p * NA_PAD + w * AT_W, AT_W)], tmp_b, sem_r)
        cp_p = pltpu.async_copy(
            pe_part.at[pl.ds(p * NA_PAD + w * AT_W, AT_W)], tmp2_b, sem_r)
        cp_r.wait()

        def addrow_r(j):
            jsl = pl.ds(j * L, L)
            rho_b[jsl] = rho_b[jsl] + tmp_b[jsl]

        _vloop(AT_ROWS, addrow_r)
        cp_p.wait()

        def addrow_p(j):
            jsl = pl.ds(j * L, L)
            pe_b[jsl] = pe_b[jsl] + tmp2_b[jsl]

        _vloop(AT_ROWS, addrow_p)

    def redwrap(p, carry):
        red(p + 1)
        return carry

    lax.fori_loop(0, NS - 1, redwrap, 0)

    def row(j):
        jsl = pl.ds(j * L, L)
        t = tb[jsl]
        rho = rho_b[jsl]
        rm = plsc.load_gather(rmin_t, [t])
        iv = plsc.load_gather(idr_t, [t])
        idxf = jnp.minimum(jnp.maximum((rho - rm) * iv, 0.0), RHO_CLIP_HI)
        idx = idxf.astype(i32)
        frac = idxf - idx.astype(f32)
        eb = t * N_RHO + idx
        F0 = plsc.load_gather(embed_t, [eb])
        F1 = plsc.load_gather(embed_t, [eb + 1])
        G0 = plsc.load_gather(embedp_t, [eb])
        G1 = plsc.load_gather(embedp_t, [eb + 1])
        en_b[jsl] = F0 + frac * (F1 - F0) + pe_b[jsl]
        fp_b[jsl] = G0 + frac * (G1 - G0)

    _vloop(AT_ROWS, row)
    pltpu.sync_copy(en_b, en_out.at[base])
    pltpu.sync_copy(fp_b, fp_out.at[base])


def _pair_pass2_body(r1, s1, d1, dbi, dbj, fp_pad, dpflat, ppflat, en_pad,
                     out1,
                     fp_s, densp_t, pairp_t,
                     r_b, s_b, d_b, bi_b, bj_b, fs_b, fd_b, f_b,
                     sem_in, sem_st, sem_out):
    c = lax.axis_index("c")
    s = lax.axis_index("s")
    w = c * NS + s

    sl = pl.ds(s * AT_SC, AT_SC)
    pltpu.sync_copy(fp_pad.at[sl], fp_s.at[sl])
    pltpu.sync_copy(dpflat, densp_t)
    pltpu.sync_copy(ppflat, pairp_t)

    nch = ECHUNKS // NW + jnp.where(w < ECHUNKS % NW, 1, 0)

    def ecopy(k):
        ebase = (w + k * NW) * CHUNK_C
        pltpu.sync_copy(en_pad.at[pl.ds(ebase, CHUNK_C)], f_b)
        pltpu.sync_copy(f_b, out1.at[pl.ds(ebase, CHUNK_C)])

    _vloop(nch, ecopy)
    plsc.subcore_barrier()

    def chunk(ci):
        base = w * PAIRS_W + ci * CHUNK_C
        cp_r = pltpu.async_copy(r1.at[pl.ds(base, CHUNK_C)], r_b, sem_in)
        cp_s = pltpu.async_copy(s1.at[pl.ds(base, CHUNK_C)], s_b, sem_in)
        cp_d = pltpu.async_copy(d1.at[pl.ds(base, CHUNK_C)], d_b, sem_in)
        cp_i = pltpu.async_copy(dbi.at[pl.ds(base, CHUNK_C)], bi_b, sem_in)
        cp_j = pltpu.async_copy(dbj.at[pl.ds(base, CHUNK_C)], bj_b, sem_in)
        cp_r.wait()
        cp_s.wait()
        cp_d.wait()
        cp_i.wait()
        cp_j.wait()
        g_s = pltpu.async_copy(fp_s.at[s_b], fs_b, sem_st)
        g_d = pltpu.async_copy(fp_s.at[d_b], fd_b, sem_st)
        g_s.wait()
        g_d.wait()

        def row(i):
            rsl = pl.ds(i * L, L)
            idx, frac = _rbin(r_b[rsl])
            bi = bi_b[rsl]
            bj = bj_b[rsl]
            pb = bi * E_TYPES + bj + idx
            p0 = plsc.load_gather(pairp_t, [pb])
            p1 = plsc.load_gather(pairp_t, [pb + 1])
            phip = p0 + frac * (p1 - p0)
            j0 = plsc.load_gather(densp_t, [bj + idx])
            j1 = plsc.load_gather(densp_t, [bj + idx + 1])
            rhop_j = j0 + frac * (j1 - j0)
            q0 = plsc.load_gather(densp_t, [bi + idx])
            q1 = plsc.load_gather(densp_t, [bi + idx + 1])
            rhop_i = q0 + frac * (q1 - q0)
            f_b[rsl] = phip + fs_b[rsl] * rhop_j + fd_b[rsl] * rhop_i

        _vloop(VECS_C, row)
        pltpu.sync_copy(f_b, out1.at[pl.ds(N_ATOMS + base, CHUNK_C)])

    _vloop(NCHUNK_C, chunk)


@functools.cache
def _build(interpret=False):
    def mesh():
        return plsc.VectorSubcoreMesh(core_axis_name="c",
                                      subcore_axis_name="s")

    params = pltpu.CompilerParams(needs_layout_passes=False)

    pass1 = pl.kernel(
        _pair_pass1_body,
        out_type=(
            jax.ShapeDtypeStruct((NS * NA_PAD,), f32),
            jax.ShapeDtypeStruct((NS * NA_PAD,), f32),
            jax.ShapeDtypeStruct((N_PAIRS,), i32),
            jax.ShapeDtypeStruct((N_PAIRS,), i32),
        ),
        mesh=mesh(),
        interpret=interpret,
        compiler_params=params,
        scratch_types=[
            pltpu.VMEM_SHARED((NA_PAD,), i32),
            pltpu.VMEM_SHARED((DENS_N,), f32),
            pltpu.VMEM_SHARED((PAIR_N,), f32),
            pltpu.VMEM((NA_PAD,), f32),
            pltpu.VMEM((CHUNK_A,), f32),
            pltpu.VMEM((CHUNK_A,), i32),
            pltpu.VMEM((CHUNK_A,), i32),
            pltpu.VMEM((CHUNK_A,), i32),
            pltpu.VMEM((CHUNK_A,), i32),
            pltpu.VMEM((CHUNK_A,), i32),
            pltpu.VMEM((CHUNK_A,), i32),
            pltpu.VMEM((CHUNK_A,), f32),
            pltpu.VMEM((CHUNK_A,), f32),
            pltpu.SemaphoreType.DMA,
            pltpu.SemaphoreType.DMA,
            pltpu.SemaphoreType.DMA,
        ],
    )

    pass_b = pl.kernel(
        _atom_pass_body,
        out_type=(
            jax.ShapeDtypeStruct((NA_PAD,), f32),
            jax.ShapeDtypeStruct((NA_PAD,), f32),
        ),
        mesh=mesh(),
        interpret=interpret,
        compiler_params=params,
        scratch_types=[
            pltpu.VMEM((E_TYPES * N_RHO,), f32),
            pltpu.VMEM((E_TYPES * N_RHO,), f32),
            pltpu.VMEM((L,), f32),
            pltpu.VMEM((L,), f32),
            pltpu.VMEM((AT_W,), f32),
            pltpu.VMEM((AT_W,), f32),
            pltpu.VMEM((AT_W,), f32),
            pltpu.VMEM((AT_W,), f32),
            pltpu.VMEM((AT_W,), i32),
            pltpu.VMEM((AT_W,), f32),
            pltpu.VMEM((AT_W,), f32),
            pltpu.SemaphoreType.DMA,
        ],
    )

    pass2 = pl.kernel(
        _pair_pass2_body,
        out_type=jax.ShapeDtypeStruct((OUT_LEN,), f32),
        mesh=mesh(),
        interpret=interpret,
        compiler_params=params,
        scratch_types=[
            pltpu.VMEM_SHARED((NA_PAD,), f32),
            pltpu.VMEM((DENS_N,), f32),
            pltpu.VMEM((PAIR_N,), f32),
            pltpu.VMEM((CHUNK_C,), f32),
            pltpu.VMEM((CHUNK_C,), i32),
            pltpu.VMEM((CHUNK_C,), i32),
            pltpu.VMEM((CHUNK_C,), i32),
            pltpu.VMEM((CHUNK_C,), i32),
            pltpu.VMEM((CHUNK_C,), f32),
            pltpu.VMEM((CHUNK_C,), f32),
            pltpu.VMEM((CHUNK_C,), f32),
            pltpu.SemaphoreType.DMA,
            pltpu.SemaphoreType.DMA,
            pltpu.SemaphoreType.DMA,
        ],
    )
    return pass1, pass_b, pass2


def _run(r, edge_index, atom_type_indices, density_table,
         density_deriv_table, pair_table, pair_deriv_table,
         embed_table, embed_deriv_table, embed_rho_min, embed_inv_drho,
         interpret=False):
    pass1, pass_b, pass2 = _build(interpret)
    src1 = edge_index[0]
    dst1 = edge_index[1]
    tpad = jnp.pad(atom_type_indices, (0, NA_PAD - N_ATOMS))
    rmin16 = jnp.pad(embed_rho_min, (0, L - E_TYPES))
    idr16 = jnp.pad(embed_inv_drho, (0, L - E_TYPES))

    rho_part, pe_part, dbi, dbj = pass1(
        r, src1, dst1, tpad,
        density_table.reshape(-1), pair_table.reshape(-1))
    en_pad, fp_pad = pass_b(
        rho_part, pe_part, tpad,
        embed_table.reshape(-1), embed_deriv_table.reshape(-1),
        rmin16, idr16)
    return pass2(
        r, src1, dst1, dbi, dbj, fp_pad,
        density_deriv_table.reshape(-1), pair_deriv_table.reshape(-1),
        en_pad)


def kernel(r, edge_index, atom_type_indices, density_table,
           density_deriv_table, pair_table, pair_deriv_table,
           embed_table, embed_deriv_table, embed_rho_min, embed_inv_drho):
    return _run(r, edge_index, atom_type_indices, density_table,
                density_deriv_table, pair_table, pair_deriv_table,
                embed_table, embed_deriv_table, embed_rho_min,
                embed_inv_drho)


# speedup vs baseline: 2.2121x; 1.2501x over previous
import functools

import jax
import jax.numpy as jnp
from jax import lax
from jax.experimental import pallas as pl
from jax.experimental.pallas import tpu as pltpu
from jax.experimental.pallas import tpu_sc as plsc

N_ATOMS = 100000
N_PAIRS = 3200000
E_TYPES = 3
N_R = 8192
N_RHO = 4096
R_MAX = 6.0
INV_DR = (N_R - 1) / R_MAX
RMAX_C = R_MAX * (1.0 - 1e-07)
RHO_CLIP_HI = N_RHO - 1 - 1e-04

NC = 2
NS = 16
NW = NC * NS
L = 16

NA_PAD = 100352
AT_W = NA_PAD // NW
AT_ROWS = AT_W // L
AT_SC = NA_PAD // NS

PAIRS_T = N_PAIRS // NS
CHUNK_A = 800
VECS_A = CHUNK_A // L
NCHUNK_A = PAIRS_T // CHUNK_A

PAIRS_W = N_PAIRS // NW
CHUNK_C = 2000
VECS_C = CHUNK_C // L
NCHUNK_C = PAIRS_W // CHUNK_C

DENS_N = E_TYPES * N_R
PAIR_N = E_TYPES * E_TYPES * N_R
OUT_LEN = N_ATOMS + N_PAIRS
ECHUNKS = N_ATOMS // CHUNK_C

f32 = jnp.float32
i32 = jnp.int32


def _rbin(rr):
    rc = jnp.minimum(jnp.maximum(rr, 0.0), RMAX_C)
    idxf = rc * INV_DR
    idx = idxf.astype(i32)
    frac = idxf - idx.astype(f32)
    return idx, frac


def _vloop(n, body):
    def wrap(i, carry):
        body(i)
        return carry
    lax.fori_loop(0, n, wrap, 0)


def _zero_fill(ref, nwords):
    zeros16 = jnp.zeros((L,), f32)

    def zf(i):
        ref[pl.ds(i * L, L)] = zeros16

    _vloop(nwords // L, zf)


def _pair_pass1_body(r1, s1, d1, tpad, dflat, pflat,
                     rho_out, pe_out, dbi_out, dbj_out,
                     type_s, dens_ts, pair_ts, acc,
                     r_b0, s_b0, d_b0, r_b1, s_b1, d_b1,
                     ti_b, tj_b, i0_b, i1_b, v0_b, v1_b,
                     sem_in, sem_t, sem_v, sem_o):
    c = lax.axis_index("c")
    s = lax.axis_index("s")

    sl = pl.ds(s * AT_SC, AT_SC)
    pltpu.sync_copy(tpad.at[sl], type_s.at[sl])
    dsl = pl.ds(s * (DENS_N // NS), DENS_N // NS)
    pltpu.sync_copy(dflat.at[dsl], dens_ts.at[dsl])
    psl = pl.ds(s * (PAIR_N // NS), PAIR_N // NS)
    pltpu.sync_copy(pflat.at[psl], pair_ts.at[psl])
    _zero_fill(acc, NA_PAD)
    plsc.subcore_barrier()

    bufs = ((r_b0, s_b0, d_b0), (r_b1, s_b1, d_b1))

    def issue_stage(k, bset):
        base = s * PAIRS_T + k * CHUNK_A
        pltpu.async_copy(r1.at[pl.ds(base, CHUNK_A)], bset[0], sem_in)
        pltpu.async_copy(s1.at[pl.ds(base, CHUNK_A)], bset[1], sem_in)
        pltpu.async_copy(d1.at[pl.ds(base, CHUNK_A)], bset[2], sem_in)

    def wait_stage(bset):
        pltpu.make_async_copy(r1.at[pl.ds(0, CHUNK_A)], bset[0],
                              sem_in).wait()
        pltpu.make_async_copy(s1.at[pl.ds(0, CHUNK_A)], bset[1],
                              sem_in).wait()
        pltpu.make_async_copy(d1.at[pl.ds(0, CHUNK_A)], bset[2],
                              sem_in).wait()

    @pl.when(c == 0)
    def _():
        def issue_types(bset):
            pltpu.async_copy(type_s.at[bset[1]], ti_b, sem_t)
            pltpu.async_copy(type_s.at[bset[2]], tj_b, sem_t)

        def wait_types(bset):
            pltpu.make_async_copy(type_s.at[bset[1]], ti_b, sem_t).wait()
            pltpu.make_async_copy(type_s.at[bset[2]], tj_b, sem_t).wait()

        def subbody(k, cur, nxt):
            next_k = k + 1

            @pl.when(next_k < NCHUNK_A)
            def _():
                issue_stage(next_k, nxt)

            wait_types(cur)

            def idxrow(i):
                rsl = pl.ds(i * L, L)
                idx, _ = _rbin(cur[0][rsl])
                dbi = ti_b[rsl] * N_R
                dbj = tj_b[rsl] * N_R
                fi = dbi * E_TYPES + dbj + idx
                i0_b[rsl] = fi
                i1_b[rsl] = fi + 1
                ti_b[rsl] = dbi
                tj_b[rsl] = dbj

            _vloop(VECS_A, idxrow)
            base = s * PAIRS_T + k * CHUNK_A
            pltpu.async_copy(pair_ts.at[i0_b], v0_b, sem_v)
            pltpu.async_copy(pair_ts.at[i1_b], v1_b, sem_v)
            pltpu.async_copy(ti_b, dbi_out.at[pl.ds(base, CHUNK_A)], sem_o)
            pltpu.async_copy(tj_b, dbj_out.at[pl.ds(base, CHUNK_A)], sem_o)

            def wait_o():
                pltpu.make_async_copy(
                    ti_b, dbi_out.at[pl.ds(0, CHUNK_A)], sem_o).wait()
                pltpu.make_async_copy(
                    tj_b, dbj_out.at[pl.ds(0, CHUNK_A)], sem_o).wait()

            @pl.when(next_k < NCHUNK_A)
            def _():
                wait_stage(nxt)
                wait_o()
                issue_types(nxt)

            @pl.when(next_k >= NCHUNK_A)
            def _():
                wait_o()

            pltpu.make_async_copy(pair_ts.at[i0_b], v0_b, sem_v).wait()
            pltpu.make_async_copy(pair_ts.at[i1_b], v1_b, sem_v).wait()

            def accrow(i):
                rsl = pl.ds(i * L, L)
                _, frac = _rbin(cur[0][rsl])
                v0 = v0_b[rsl]
                phi = v0 + frac * (v1_b[rsl] - v0)
                plsc.addupdate_scatter(acc, [cur[1][rsl]], 0.5 * phi)

            _vloop(VECS_A, accrow)

        issue_stage(0, bufs[0])
        wait_stage(bufs[0])
        issue_types(bufs[0])

        def pair_iter(kk):
            subbody(2 * kk, bufs[0], bufs[1])
            subbody(2 * kk + 1, bufs[1], bufs[0])

        _vloop(NCHUNK_A // 2, pair_iter)
        pltpu.sync_copy(acc, pe_out.at[pl.ds(s * NA_PAD, NA_PAD)])

    @pl.when(c == 1)
    def _():
        def issue_types(bset):
            pltpu.async_copy(type_s.at[bset[2]], tj_b, sem_t)

        def wait_types(bset):
            pltpu.make_async_copy(type_s.at[bset[2]], tj_b, sem_t).wait()

        def subbody(k, cur, nxt):
            next_k = k + 1

            @pl.when(next_k < NCHUNK_A)
            def _():
                issue_stage(next_k, nxt)

            wait_types(cur)

            def idxrow(i):
                rsl = pl.ds(i * L, L)
                idx, _ = _rbin(cur[0][rsl])
                fi = tj_b[rsl] * N_R + idx
                i0_b[rsl] = fi
                i1_b[rsl] = fi + 1

            _vloop(VECS_A, idxrow)
            pltpu.async_copy(dens_ts.at[i0_b], v0_b, sem_v)
            pltpu.async_copy(dens_ts.at[i1_b], v1_b, sem_v)

            @pl.when(next_k < NCHUNK_A)
            def _():
                wait_stage(nxt)
                issue_types(nxt)

            pltpu.make_async_copy(dens_ts.at[i0_b], v0_b, sem_v).wait()
            pltpu.make_async_copy(dens_ts.at[i1_b], v1_b, sem_v).wait()

            def accrow(i):
                rsl = pl.ds(i * L, L)
                _, frac = _rbin(cur[0][rsl])
                v0 = v0_b[rsl]
                dens = v0 + frac * (v1_b[rsl] - v0)
                plsc.addupdate_scatter(acc, [cur[1][rsl]], dens)

            _vloop(VECS_A, accrow)

        issue_stage(0, bufs[0])
        wait_stage(bufs[0])
        issue_types(bufs[0])

        def pair_iter(kk):
            subbody(2 * kk, bufs[0], bufs[1])
            subbody(2 * kk + 1, bufs[1], bufs[0])

        _vloop(NCHUNK_A // 2, pair_iter)
        pltpu.sync_copy(acc, rho_out.at[pl.ds(s * NA_PAD, NA_PAD)])


def _atom_pass_body(rho_part, pe_part, tpad, eflat, epflat, rmin16, idr16,
                    en_out, fp_out,
                    embed_t, embedp_t, rmin_t, idr_t,
                    rho_b, pe_b, tmp_b, tmp2_b, tb, en_b, fp_b, sem_r):
    c = lax.axis_index("c")
    s = lax.axis_index("s")
    w = c * NS + s
    base = pl.ds(w * AT_W, AT_W)

    pltpu.sync_copy(eflat, embed_t)
    pltpu.sync_copy(epflat, embedp_t)
    pltpu.sync_copy(rmin16, rmin_t)
    pltpu.sync_copy(idr16, idr_t)
    pltpu.sync_copy(tpad.at[base], tb)

    pltpu.sync_copy(rho_part.at[pl.ds(w * AT_W, AT_W)], rho_b)
    pltpu.sync_copy(pe_part.at[pl.ds(w * AT_W, AT_W)], pe_b)

    def red(p):
        cp_r = pltpu.async_copy(
            rho_part.at[pl.ds(p * NA_PAD + w * AT_W, AT_W)], tmp_b, sem_r)
        cp_p = pltpu.async_copy(
            pe_part.at[pl.ds(p * NA_PAD + w * AT_W, AT_W)], tmp2_b, sem_r)
        cp_r.wait()

        def addrow_r(j):
            jsl = pl.ds(j * L, L)
            rho_b[jsl] = rho_b[jsl] + tmp_b[jsl]

        _vloop(AT_ROWS, addrow_r)
        cp_p.wait()

        def addrow_p(j):
            jsl = pl.ds(j * L, L)
            pe_b[jsl] = pe_b[jsl] + tmp2_b[jsl]

        _vloop(AT_ROWS, addrow_p)

    def redwrap(p, carry):
        red(p + 1)
        return carry

    lax.fori_loop(0, NS - 1, redwrap, 0)

    def row(j):
        jsl = pl.ds(j * L, L)
        t = tb[jsl]
        rho = rho_b[jsl]
        rm = plsc.load_gather(rmin_t, [t])
        iv = plsc.load_gather(idr_t, [t])
        idxf = jnp.minimum(jnp.maximum((rho - rm) * iv, 0.0), RHO_CLIP_HI)
        idx = idxf.astype(i32)
        frac = idxf - idx.astype(f32)
        eb = t * N_RHO + idx
        F0 = plsc.load_gather(embed_t, [eb])
        F1 = plsc.load_gather(embed_t, [eb + 1])
        G0 = plsc.load_gather(embedp_t, [eb])
        G1 = plsc.load_gather(embedp_t, [eb + 1])
        en_b[jsl] = F0 + frac * (F1 - F0) + pe_b[jsl]
        fp_b[jsl] = G0 + frac * (G1 - G0)

    _vloop(AT_ROWS, row)
    pltpu.sync_copy(en_b, en_out.at[base])
    pltpu.sync_copy(fp_b, fp_out.at[base])


def _pair_pass2_body(r1, s1, d1, dbi, dbj, fp_pad, dpflat, ppflat, en_pad,
                     out1,
                     fp_s, densp_t, pairp_t,
                     r_b, s_b, d_b, bi_b, bj_b, fs_b, fd_b, f_b,
                     sem_in, sem_st, sem_out):
    c = lax.axis_index("c")
    s = lax.axis_index("s")
    w = c * NS + s

    sl = pl.ds(s * AT_SC, AT_SC)
    pltpu.sync_copy(fp_pad.at[sl], fp_s.at[sl])
    pltpu.sync_copy(dpflat, densp_t)
    pltpu.sync_copy(ppflat, pairp_t)

    nch = ECHUNKS // NW + jnp.where(w < ECHUNKS % NW, 1, 0)

    def ecopy(k):
        ebase = (w + k * NW) * CHUNK_C
        pltpu.sync_copy(en_pad.at[pl.ds(ebase, CHUNK_C)], f_b)
        pltpu.sync_copy(f_b, out1.at[pl.ds(ebase, CHUNK_C)])

    _vloop(nch, ecopy)
    plsc.subcore_barrier()

    def chunk(ci):
        base = w * PAIRS_W + ci * CHUNK_C
        cp_r = pltpu.async_copy(r1.at[pl.ds(base, CHUNK_C)], r_b, sem_in)
        cp_s = pltpu.async_copy(s1.at[pl.ds(base, CHUNK_C)], s_b, sem_in)
        cp_d = pltpu.async_copy(d1.at[pl.ds(base, CHUNK_C)], d_b, sem_in)
        cp_i = pltpu.async_copy(dbi.at[pl.ds(base, CHUNK_C)], bi_b, sem_in)
        cp_j = pltpu.async_copy(dbj.at[pl.ds(base, CHUNK_C)], bj_b, sem_in)
        cp_r.wait()
        cp_s.wait()
        cp_d.wait()
        cp_i.wait()
        cp_j.wait()
        g_s = pltpu.async_copy(fp_s.at[s_b], fs_b, sem_st)
        g_d = pltpu.async_copy(fp_s.at[d_b], fd_b, sem_st)
        g_s.wait()
        g_d.wait()

        def row(i):
            rsl = pl.ds(i * L, L)
            idx, frac = _rbin(r_b[rsl])
            bi = bi_b[rsl]
            bj = bj_b[rsl]
            pb = bi * E_TYPES + bj + idx
            p0 = plsc.load_gather(pairp_t, [pb])
            p1 = plsc.load_gather(pairp_t, [pb + 1])
            phip = p0 + frac * (p1 - p0)
            j0 = plsc.load_gather(densp_t, [bj + idx])
            j1 = plsc.load_gather(densp_t, [bj + idx + 1])
            rhop_j = j0 + frac * (j1 - j0)
            q0 = plsc.load_gather(densp_t, [bi + idx])
            q1 = plsc.load_gather(densp_t, [bi + idx + 1])
            rhop_i = q0 + frac * (q1 - q0)
            f_b[rsl] = phip + fs_b[rsl] * rhop_j + fd_b[rsl] * rhop_i

        _vloop(VECS_C, row)
        pltpu.sync_copy(f_b, out1.at[pl.ds(N_ATOMS + base, CHUNK_C)])

    _vloop(NCHUNK_C, chunk)


@functools.cache
def _build(interpret=False):
    def mesh():
        return plsc.VectorSubcoreMesh(core_axis_name="c",
                                      subcore_axis_name="s")

    params = pltpu.CompilerParams(needs_layout_passes=False)

    pass1 = pl.kernel(
        _pair_pass1_body,
        out_type=(
            jax.ShapeDtypeStruct((NS * NA_PAD,), f32),
            jax.ShapeDtypeStruct((NS * NA_PAD,), f32),
            jax.ShapeDtypeStruct((N_PAIRS,), i32),
            jax.ShapeDtypeStruct((N_PAIRS,), i32),
        ),
        mesh=mesh(),
        interpret=interpret,
        compiler_params=params,
        scratch_types=[
            pltpu.VMEM_SHARED((NA_PAD,), i32),
            pltpu.VMEM_SHARED((DENS_N,), f32),
            pltpu.VMEM_SHARED((PAIR_N,), f32),
            pltpu.VMEM((NA_PAD,), f32),
            pltpu.VMEM((CHUNK_A,), f32),
            pltpu.VMEM((CHUNK_A,), i32),
            pltpu.VMEM((CHUNK_A,), i32),
            pltpu.VMEM((CHUNK_A,), f32),
            pltpu.VMEM((CHUNK_A,), i32),
            pltpu.VMEM((CHUNK_A,), i32),
            pltpu.VMEM((CHUNK_A,), i32),
            pltpu.VMEM((CHUNK_A,), i32),
            pltpu.VMEM((CHUNK_A,), i32),
            pltpu.VMEM((CHUNK_A,), i32),
            pltpu.VMEM((CHUNK_A,), f32),
            pltpu.VMEM((CHUNK_A,), f32),
            pltpu.SemaphoreType.DMA,
            pltpu.SemaphoreType.DMA,
            pltpu.SemaphoreType.DMA,
            pltpu.SemaphoreType.DMA,
        ],
    )

    pass_b = pl.kernel(
        _atom_pass_body,
        out_type=(
            jax.ShapeDtypeStruct((NA_PAD,), f32),
            jax.ShapeDtypeStruct((NA_PAD,), f32),
        ),
        mesh=mesh(),
        interpret=interpret,
        compiler_params=params,
        scratch_types=[
            pltpu.VMEM((E_TYPES * N_RHO,), f32),
            pltpu.VMEM((E_TYPES * N_RHO,), f32),
            pltpu.VMEM((L,), f32),
            pltpu.VMEM((L,), f32),
            pltpu.VMEM((AT_W,), f32),
            pltpu.VMEM((AT_W,), f32),
            pltpu.VMEM((AT_W,), f32),
            pltpu.VMEM((AT_W,), f32),
            pltpu.VMEM((AT_W,), i32),
            pltpu.VMEM((AT_W,), f32),
            pltpu.VMEM((AT_W,), f32),
            pltpu.SemaphoreType.DMA,
        ],
    )

    pass2 = pl.kernel(
        _pair_pass2_body,
        out_type=jax.ShapeDtypeStruct((OUT_LEN,), f32),
        mesh=mesh(),
        interpret=interpret,
        compiler_params=params,
        scratch_types=[
            pltpu.VMEM_SHARED((NA_PAD,), f32),
            pltpu.VMEM((DENS_N,), f32),
            pltpu.VMEM((PAIR_N,), f32),
            pltpu.VMEM((CHUNK_C,), f32),
            pltpu.VMEM((CHUNK_C,), i32),
            pltpu.VMEM((CHUNK_C,), i32),
            pltpu.VMEM((CHUNK_C,), i32),
            pltpu.VMEM((CHUNK_C,), i32),
            pltpu.VMEM((CHUNK_C,), f32),
            pltpu.VMEM((CHUNK_C,), f32),
            pltpu.VMEM((CHUNK_C,), f32),
            pltpu.SemaphoreType.DMA,
            pltpu.SemaphoreType.DMA,
            pltpu.SemaphoreType.DMA,
        ],
    )
    return pass1, pass_b, pass2


def _run(r, edge_index, atom_type_indices, density_table,
         density_deriv_table, pair_table, pair_deriv_table,
         embed_table, embed_deriv_table, embed_rho_min, embed_inv_drho,
         interpret=False):
    pass1, pass_b, pass2 = _build(interpret)
    src1 = edge_index[0]
    dst1 = edge_index[1]
    tpad = jnp.pad(atom_type_indices, (0, NA_PAD - N_ATOMS))
    rmin16 = jnp.pad(embed_rho_min, (0, L - E_TYPES))
    idr16 = jnp.pad(embed_inv_drho, (0, L - E_TYPES))

    rho_part, pe_part, dbi, dbj = pass1(
        r, src1, dst1, tpad,
        density_table.reshape(-1), pair_table.reshape(-1))
    en_pad, fp_pad = pass_b(
        rho_part, pe_part, tpad,
        embed_table.reshape(-1), embed_deriv_table.reshape(-1),
        rmin16, idr16)
    return pass2(
        r, src1, dst1, dbi, dbj, fp_pad,
        density_deriv_table.reshape(-1), pair_deriv_table.reshape(-1),
        en_pad)


def kernel(r, edge_index, atom_type_indices, density_table,
           density_deriv_table, pair_table, pair_deriv_table,
           embed_table, embed_deriv_table, embed_rho_min, embed_inv_drho):
    return _run(r, edge_index, atom_type_indices, density_table,
                density_deriv_table, pair_table, pair_deriv_table,
                embed_table, embed_deriv_table, embed_rho_min,
                embed_inv_drho)


# speedup vs baseline: 2.2741x; 1.0280x over previous
import functools

import jax
import jax.numpy as jnp
from jax import lax
from jax.experimental import pallas as pl
from jax.experimental.pallas import tpu as pltpu
from jax.experimental.pallas import tpu_sc as plsc

N_ATOMS = 100000
N_PAIRS = 3200000
E_TYPES = 3
N_R = 8192
N_RHO = 4096
R_MAX = 6.0
INV_DR = (N_R - 1) / R_MAX
RMAX_C = R_MAX * (1.0 - 1e-07)
RHO_CLIP_HI = N_RHO - 1 - 1e-04

NC = 2
NS = 16
NW = NC * NS
L = 16

NA_PAD = 100352
AT_W = NA_PAD // NW
AT_ROWS = AT_W // L
AT_SC = NA_PAD // NS

PAIRS_T = N_PAIRS // NS
CHUNK_A = 800
VECS_A = CHUNK_A // L
NCHUNK_A = PAIRS_T // CHUNK_A

PAIRS_W = N_PAIRS // NW
CHUNK_C = 800
VECS_C = CHUNK_C // L
NCHUNK_C = PAIRS_W // CHUNK_C

DENS_N = E_TYPES * N_R
PAIR_N = E_TYPES * E_TYPES * N_R
OUT_LEN = N_ATOMS + N_PAIRS
ECHUNKS = N_ATOMS // CHUNK_C

f32 = jnp.float32
i32 = jnp.int32


def _rbin(rr):
    rc = jnp.minimum(jnp.maximum(rr, 0.0), RMAX_C)
    idxf = rc * INV_DR
    idx = idxf.astype(i32)
    frac = idxf - idx.astype(f32)
    return idx, frac


def _vloop(n, body):
    def wrap(i, carry):
        body(i)
        return carry
    lax.fori_loop(0, n, wrap, 0)


def _zero_fill(ref, nwords):
    zeros16 = jnp.zeros((L,), f32)

    def zf(i):
        ref[pl.ds(i * L, L)] = zeros16

    _vloop(nwords // L, zf)


def _pair_pass1_body(r1, s1, d1, tpad, dflat, pflat,
                     rho_out, pe_out, dbi_out, dbj_out,
                     type_s, dens_ts, pair_ts, acc,
                     r_b0, s_b0, d_b0, r_b1, s_b1, d_b1,
                     ti_b, tj_b, i0_b, i1_b, v0_b, v1_b,
                     sem_in, sem_t, sem_v, sem_o):
    c = lax.axis_index("c")
    s = lax.axis_index("s")

    sl = pl.ds(s * AT_SC, AT_SC)
    pltpu.sync_copy(tpad.at[sl], type_s.at[sl])
    dsl = pl.ds(s * (DENS_N // NS), DENS_N // NS)
    pltpu.sync_copy(dflat.at[dsl], dens_ts.at[dsl])
    psl = pl.ds(s * (PAIR_N // NS), PAIR_N // NS)
    pltpu.sync_copy(pflat.at[psl], pair_ts.at[psl])
    _zero_fill(acc, NA_PAD)
    plsc.subcore_barrier()

    bufs = ((r_b0, s_b0, d_b0), (r_b1, s_b1, d_b1))

    def issue_stage(k, bset):
        base = s * PAIRS_T + k * CHUNK_A
        pltpu.async_copy(r1.at[pl.ds(base, CHUNK_A)], bset[0], sem_in)
        pltpu.async_copy(s1.at[pl.ds(base, CHUNK_A)], bset[1], sem_in)
        pltpu.async_copy(d1.at[pl.ds(base, CHUNK_A)], bset[2], sem_in)

    def wait_stage(bset):
        pltpu.make_async_copy(r1.at[pl.ds(0, CHUNK_A)], bset[0],
                              sem_in).wait()
        pltpu.make_async_copy(s1.at[pl.ds(0, CHUNK_A)], bset[1],
                              sem_in).wait()
        pltpu.make_async_copy(d1.at[pl.ds(0, CHUNK_A)], bset[2],
                              sem_in).wait()

    @pl.when(c == 0)
    def _():
        def issue_types(bset):
            pltpu.async_copy(type_s.at[bset[1]], ti_b, sem_t)
            pltpu.async_copy(type_s.at[bset[2]], tj_b, sem_t)

        def wait_types(bset):
            pltpu.make_async_copy(type_s.at[bset[1]], ti_b, sem_t).wait()
            pltpu.make_async_copy(type_s.at[bset[2]], tj_b, sem_t).wait()

        def subbody(k, cur, nxt):
            next_k = k + 1

            @pl.when(next_k < NCHUNK_A)
            def _():
                issue_stage(next_k, nxt)

            wait_types(cur)

            def idxrow(i):
                rsl = pl.ds(i * L, L)
                idx, _ = _rbin(cur[0][rsl])
                dbi = ti_b[rsl] * N_R
                dbj = tj_b[rsl] * N_R
                fi = dbi * E_TYPES + dbj + idx
                i0_b[rsl] = fi
                i1_b[rsl] = fi + 1
                ti_b[rsl] = dbi
                tj_b[rsl] = dbj

            _vloop(VECS_A, idxrow)
            base = s * PAIRS_T + k * CHUNK_A
            pltpu.async_copy(pair_ts.at[i0_b], v0_b, sem_v)
            pltpu.async_copy(pair_ts.at[i1_b], v1_b, sem_v)
            pltpu.async_copy(ti_b, dbi_out.at[pl.ds(base, CHUNK_A)], sem_o)
            pltpu.async_copy(tj_b, dbj_out.at[pl.ds(base, CHUNK_A)], sem_o)

            def wait_o():
                pltpu.make_async_copy(
                    ti_b, dbi_out.at[pl.ds(0, CHUNK_A)], sem_o).wait()
                pltpu.make_async_copy(
                    tj_b, dbj_out.at[pl.ds(0, CHUNK_A)], sem_o).wait()

            @pl.when(next_k < NCHUNK_A)
            def _():
                wait_stage(nxt)
                wait_o()
                issue_types(nxt)

            @pl.when(next_k >= NCHUNK_A)
            def _():
                wait_o()

            pltpu.make_async_copy(pair_ts.at[i0_b], v0_b, sem_v).wait()
            pltpu.make_async_copy(pair_ts.at[i1_b], v1_b, sem_v).wait()

            def accrow(i):
                rsl = pl.ds(i * L, L)
                _, frac = _rbin(cur[0][rsl])
                v0 = v0_b[rsl]
                phi = v0 + frac * (v1_b[rsl] - v0)
                plsc.addupdate_scatter(acc, [cur[1][rsl]], 0.5 * phi)

            _vloop(VECS_A, accrow)

        issue_stage(0, bufs[0])
        wait_stage(bufs[0])
        issue_types(bufs[0])

        def pair_iter(kk):
            subbody(2 * kk, bufs[0], bufs[1])
            subbody(2 * kk + 1, bufs[1], bufs[0])

        _vloop(NCHUNK_A // 2, pair_iter)
        pltpu.sync_copy(acc, pe_out.at[pl.ds(s * NA_PAD, NA_PAD)])

    @pl.when(c == 1)
    def _():
        def issue_types(bset):
            pltpu.async_copy(type_s.at[bset[2]], tj_b, sem_t)

        def wait_types(bset):
            pltpu.make_async_copy(type_s.at[bset[2]], tj_b, sem_t).wait()

        def subbody(k, cur, nxt):
            next_k = k + 1

            @pl.when(next_k < NCHUNK_A)
            def _():
                issue_stage(next_k, nxt)

            wait_types(cur)

            def idxrow(i):
                rsl = pl.ds(i * L, L)
                idx, _ = _rbin(cur[0][rsl])
                fi = tj_b[rsl] * N_R + idx
                i0_b[rsl] = fi
                i1_b[rsl] = fi + 1

            _vloop(VECS_A, idxrow)
            pltpu.async_copy(dens_ts.at[i0_b], v0_b, sem_v)
            pltpu.async_copy(dens_ts.at[i1_b], v1_b, sem_v)

            @pl.when(next_k < NCHUNK_A)
            def _():
                wait_stage(nxt)
                issue_types(nxt)

            pltpu.make_async_copy(dens_ts.at[i0_b], v0_b, sem_v).wait()
            pltpu.make_async_copy(dens_ts.at[i1_b], v1_b, sem_v).wait()

            def accrow(i):
                rsl = pl.ds(i * L, L)
                _, frac = _rbin(cur[0][rsl])
                v0 = v0_b[rsl]
                dens = v0 + frac * (v1_b[rsl] - v0)
                plsc.addupdate_scatter(acc, [cur[1][rsl]], dens)

            _vloop(VECS_A, accrow)

        issue_stage(0, bufs[0])
        wait_stage(bufs[0])
        issue_types(bufs[0])

        def pair_iter(kk):
            subbody(2 * kk, bufs[0], bufs[1])
            subbody(2 * kk + 1, bufs[1], bufs[0])

        _vloop(NCHUNK_A // 2, pair_iter)
        pltpu.sync_copy(acc, rho_out.at[pl.ds(s * NA_PAD, NA_PAD)])


def _atom_pass_body(rho_part, pe_part, tpad, eflat, epflat, rmin16, idr16,
                    en_out, fp_out,
                    embed_t, embedp_t, rmin_t, idr_t,
                    rho_b, pe_b, tmp_b, tmp2_b, tb, en_b, fp_b, sem_r):
    c = lax.axis_index("c")
    s = lax.axis_index("s")
    w = c * NS + s
    base = pl.ds(w * AT_W, AT_W)

    pltpu.sync_copy(eflat, embed_t)
    pltpu.sync_copy(epflat, embedp_t)
    pltpu.sync_copy(rmin16, rmin_t)
    pltpu.sync_copy(idr16, idr_t)
    pltpu.sync_copy(tpad.at[base], tb)

    pltpu.sync_copy(rho_part.at[pl.ds(w * AT_W, AT_W)], rho_b)
    pltpu.sync_copy(pe_part.at[pl.ds(w * AT_W, AT_W)], pe_b)

    def red(p):
        cp_r = pltpu.async_copy(
            rho_part.at[pl.ds(p * NA_PAD + w * AT_W, AT_W)], tmp_b, sem_r)
        cp_p = pltpu.async_copy(
            pe_part.at[pl.ds(p * NA_PAD + w * AT_W, AT_W)], tmp2_b, sem_r)
        cp_r.wait()

        def addrow_r(j):
            jsl = pl.ds(j * L, L)
            rho_b[jsl] = rho_b[jsl] + tmp_b[jsl]

        _vloop(AT_ROWS, addrow_r)
        cp_p.wait()

        def addrow_p(j):
            jsl = pl.ds(j * L, L)
            pe_b[jsl] = pe_b[jsl] + tmp2_b[jsl]

        _vloop(AT_ROWS, addrow_p)

    def redwrap(p, carry):
        red(p + 1)
        return carry

    lax.fori_loop(0, NS - 1, redwrap, 0)

    def row(j):
        jsl = pl.ds(j * L, L)
        t = tb[jsl]
        rho = rho_b[jsl]
        rm = plsc.load_gather(rmin_t, [t])
        iv = plsc.load_gather(idr_t, [t])
        idxf = jnp.minimum(jnp.maximum((rho - rm) * iv, 0.0), RHO_CLIP_HI)
        idx = idxf.astype(i32)
        frac = idxf - idx.astype(f32)
        eb = t * N_RHO + idx
        F0 = plsc.load_gather(embed_t, [eb])
        F1 = plsc.load_gather(embed_t, [eb + 1])
        G0 = plsc.load_gather(embedp_t, [eb])
        G1 = plsc.load_gather(embedp_t, [eb + 1])
        en_b[jsl] = F0 + frac * (F1 - F0) + pe_b[jsl]
        fp_b[jsl] = G0 + frac * (G1 - G0)

    _vloop(AT_ROWS, row)
    pltpu.sync_copy(en_b, en_out.at[base])
    pltpu.sync_copy(fp_b, fp_out.at[base])


def _pair_pass2_body(r1, s1, d1, dbi, dbj, fp_pad, dpflat, ppflat, en_pad,
                     out1,
                     fp_s, densp_t, pairp_t,
                     r_b0, s_b0, d_b0, bi_b0, bj_b0, fs_b0, fd_b0,
                     r_b1, s_b1, d_b1, bi_b1, bj_b1, fs_b1, fd_b1,
                     f_b0, f_b1,
                     sem_in, sem_st, sem_out):
    c = lax.axis_index("c")
    s = lax.axis_index("s")
    w = c * NS + s

    sl = pl.ds(s * AT_SC, AT_SC)
    pltpu.sync_copy(fp_pad.at[sl], fp_s.at[sl])
    pltpu.sync_copy(dpflat, densp_t)
    pltpu.sync_copy(ppflat, pairp_t)

    nch = ECHUNKS // NW + jnp.where(w < ECHUNKS % NW, 1, 0)

    def ecopy(k):
        ebase = (w + k * NW) * CHUNK_C
        pltpu.sync_copy(en_pad.at[pl.ds(ebase, CHUNK_C)], f_b0)
        pltpu.sync_copy(f_b0, out1.at[pl.ds(ebase, CHUNK_C)])

    _vloop(nch, ecopy)
    plsc.subcore_barrier()

    bufs = ((r_b0, s_b0, d_b0, bi_b0, bj_b0, fs_b0, fd_b0, f_b0),
            (r_b1, s_b1, d_b1, bi_b1, bj_b1, fs_b1, fd_b1, f_b1))

    def issue_stage(k, bset):
        base = w * PAIRS_W + k * CHUNK_C
        pltpu.async_copy(r1.at[pl.ds(base, CHUNK_C)], bset[0], sem_in)
        pltpu.async_copy(s1.at[pl.ds(base, CHUNK_C)], bset[1], sem_in)
        pltpu.async_copy(d1.at[pl.ds(base, CHUNK_C)], bset[2], sem_in)
        pltpu.async_copy(dbi.at[pl.ds(base, CHUNK_C)], bset[3], sem_in)
        pltpu.async_copy(dbj.at[pl.ds(base, CHUNK_C)], bset[4], sem_in)

    def wait_stage(bset):
        for ref, hb in ((bset[0], r1), (bset[1], s1), (bset[2], d1),
                        (bset[3], dbi), (bset[4], dbj)):
            pltpu.make_async_copy(hb.at[pl.ds(0, CHUNK_C)], ref,
                                  sem_in).wait()

    def issue_fp(bset):
        pltpu.async_copy(fp_s.at[bset[1]], bset[5], sem_st)
        pltpu.async_copy(fp_s.at[bset[2]], bset[6], sem_st)

    def wait_fp(bset):
        pltpu.make_async_copy(fp_s.at[bset[1]], bset[5], sem_st).wait()
        pltpu.make_async_copy(fp_s.at[bset[2]], bset[6], sem_st).wait()

    def subbody(k, cur, nxt):
        next_k = k + 1

        @pl.when(next_k < NCHUNK_C)
        def _():
            issue_stage(next_k, nxt)

        @pl.when(k >= 2)
        def _():
            pltpu.make_async_copy(
                cur[7], out1.at[pl.ds(N_ATOMS, CHUNK_C)], sem_out).wait()

        wait_fp(cur)

        @pl.when(next_k < NCHUNK_C)
        def _():
            wait_stage(nxt)
            issue_fp(nxt)

        def row(i):
            rsl = pl.ds(i * L, L)
            idx, frac = _rbin(cur[0][rsl])
            bi = cur[3][rsl]
            bj = cur[4][rsl]
            pb = bi * E_TYPES + bj + idx
            p0 = plsc.load_gather(pairp_t, [pb])
            p1 = plsc.load_gather(pairp_t, [pb + 1])
            phip = p0 + frac * (p1 - p0)
            j0 = plsc.load_gather(densp_t, [bj + idx])
            j1 = plsc.load_gather(densp_t, [bj + idx + 1])
            rhop_j = j0 + frac * (j1 - j0)
            q0 = plsc.load_gather(densp_t, [bi + idx])
            q1 = plsc.load_gather(densp_t, [bi + idx + 1])
            rhop_i = q0 + frac * (q1 - q0)
            cur[7][rsl] = phip + cur[5][rsl] * rhop_j + cur[6][rsl] * rhop_i

        _vloop(VECS_C, row)
        base = w * PAIRS_W + k * CHUNK_C
        pltpu.async_copy(cur[7], out1.at[pl.ds(N_ATOMS + base, CHUNK_C)],
                         sem_out)

    issue_stage(0, bufs[0])
    wait_stage(bufs[0])
    issue_fp(bufs[0])

    def pair_iter(kk):
        subbody(2 * kk, bufs[0], bufs[1])
        subbody(2 * kk + 1, bufs[1], bufs[0])

    _vloop(NCHUNK_C // 2, pair_iter)
    if NCHUNK_C % 2:
        subbody(NCHUNK_C - 1, bufs[0], bufs[1])
    pltpu.make_async_copy(f_b0, out1.at[pl.ds(N_ATOMS, CHUNK_C)],
                          sem_out).wait()
    pltpu.make_async_copy(f_b1, out1.at[pl.ds(N_ATOMS, CHUNK_C)],
                          sem_out).wait()


@functools.cache
def _build(interpret=False):
    def mesh():
        return plsc.VectorSubcoreMesh(core_axis_name="c",
                                      subcore_axis_name="s")

    params = pltpu.CompilerParams(needs_layout_passes=False)

    pass1 = pl.kernel(
        _pair_pass1_body,
        out_type=(
            jax.ShapeDtypeStruct((NS * NA_PAD,), f32),
            jax.ShapeDtypeStruct((NS * NA_PAD,), f32),
            jax.ShapeDtypeStruct((N_PAIRS,), i32),
            jax.ShapeDtypeStruct((N_PAIRS,), i32),
        ),
        mesh=mesh(),
        interpret=interpret,
        compiler_params=params,
        scratch_types=[
            pltpu.VMEM_SHARED((NA_PAD,), i32),
            pltpu.VMEM_SHARED((DENS_N,), f32),
            pltpu.VMEM_SHARED((PAIR_N,), f32),
            pltpu.VMEM((NA_PAD,), f32),
            pltpu.VMEM((CHUNK_A,), f32),
            pltpu.VMEM((CHUNK_A,), i32),
            pltpu.VMEM((CHUNK_A,), i32),
            pltpu.VMEM((CHUNK_A,), f32),
            pltpu.VMEM((CHUNK_A,), i32),
            pltpu.VMEM((CHUNK_A,), i32),
            pltpu.VMEM((CHUNK_A,), i32),
            pltpu.VMEM((CHUNK_A,), i32),
            pltpu.VMEM((CHUNK_A,), i32),
            pltpu.VMEM((CHUNK_A,), i32),
            pltpu.VMEM((CHUNK_A,), f32),
            pltpu.VMEM((CHUNK_A,), f32),
            pltpu.SemaphoreType.DMA,
            pltpu.SemaphoreType.DMA,
            pltpu.SemaphoreType.DMA,
            pltpu.SemaphoreType.DMA,
        ],
    )

    pass_b = pl.kernel(
        _atom_pass_body,
        out_type=(
            jax.ShapeDtypeStruct((NA_PAD,), f32),
            jax.ShapeDtypeStruct((NA_PAD,), f32),
        ),
        mesh=mesh(),
        interpret=interpret,
        compiler_params=params,
        scratch_types=[
            pltpu.VMEM((E_TYPES * N_RHO,), f32),
            pltpu.VMEM((E_TYPES * N_RHO,), f32),
            pltpu.VMEM((L,), f32),
            pltpu.VMEM((L,), f32),
            pltpu.VMEM((AT_W,), f32),
            pltpu.VMEM((AT_W,), f32),
            pltpu.VMEM((AT_W,), f32),
            pltpu.VMEM((AT_W,), f32),
            pltpu.VMEM((AT_W,), i32),
            pltpu.VMEM((AT_W,), f32),
            pltpu.VMEM((AT_W,), f32),
            pltpu.SemaphoreType.DMA,
        ],
    )

    pass2 = pl.kernel(
        _pair_pass2_body,
        out_type=jax.ShapeDtypeStruct((OUT_LEN,), f32),
        mesh=mesh(),
        interpret=interpret,
        compiler_params=params,
        scratch_types=[
            pltpu.VMEM_SHARED((NA_PAD,), f32),
            pltpu.VMEM((DENS_N,), f32),
            pltpu.VMEM((PAIR_N,), f32),
            pltpu.VMEM((CHUNK_C,), f32),
            pltpu.VMEM((CHUNK_C,), i32),
            pltpu.VMEM((CHUNK_C,), i32),
            pltpu.VMEM((CHUNK_C,), i32),
            pltpu.VMEM((CHUNK_C,), i32),
            pltpu.VMEM((CHUNK_C,), f32),
            pltpu.VMEM((CHUNK_C,), f32),
            pltpu.VMEM((CHUNK_C,), f32),
            pltpu.VMEM((CHUNK_C,), i32),
            pltpu.VMEM((CHUNK_C,), i32),
            pltpu.VMEM((CHUNK_C,), i32),
            pltpu.VMEM((CHUNK_C,), i32),
            pltpu.VMEM((CHUNK_C,), f32),
            pltpu.VMEM((CHUNK_C,), f32),
            pltpu.VMEM((CHUNK_C,), f32),
            pltpu.VMEM((CHUNK_C,), f32),
            pltpu.SemaphoreType.DMA,
            pltpu.SemaphoreType.DMA,
            pltpu.SemaphoreType.DMA,
        ],
    )
    return pass1, pass_b, pass2


def _run(r, edge_index, atom_type_indices, density_table,
         density_deriv_table, pair_table, pair_deriv_table,
         embed_table, embed_deriv_table, embed_rho_min, embed_inv_drho,
         interpret=False):
    pass1, pass_b, pass2 = _build(interpret)
    src1 = edge_index[0]
    dst1 = edge_index[1]
    tpad = jnp.pad(atom_type_indices, (0, NA_PAD - N_ATOMS))
    rmin16 = jnp.pad(embed_rho_min, (0, L - E_TYPES))
    idr16 = jnp.pad(embed_inv_drho, (0, L - E_TYPES))

    rho_part, pe_part, dbi, dbj = pass1(
        r, src1, dst1, tpad,
        density_table.reshape(-1), pair_table.reshape(-1))
    en_pad, fp_pad = pass_b(
        rho_part, pe_part, tpad,
        embed_table.reshape(-1), embed_deriv_table.reshape(-1),
        rmin16, idr16)
    return pass2(
        r, src1, dst1, dbi, dbj, fp_pad,
        density_deriv_table.reshape(-1), pair_deriv_table.reshape(-1),
        en_pad)


def kernel(r, edge_index, atom_type_indices, density_table,
           density_deriv_table, pair_table, pair_deriv_table,
           embed_table, embed_deriv_table, embed_rho_min, embed_inv_drho):
    return _run(r, edge_index, atom_type_indices, density_table,
                density_deriv_table, pair_table, pair_deriv_table,
                embed_table, embed_deriv_table, embed_rho_min,
                embed_inv_drho)


# speedup vs baseline: 2.7851x; 1.2247x over previous
import functools

import jax
import jax.numpy as jnp
from jax import lax
from jax.experimental import pallas as pl
from jax.experimental.pallas import tpu as pltpu
from jax.experimental.pallas import tpu_sc as plsc

N_ATOMS = 100000
N_PAIRS = 3200000
E_TYPES = 3
N_R = 8192
N_RHO = 4096
R_MAX = 6.0
INV_DR = (N_R - 1) / R_MAX
RMAX_C = R_MAX * (1.0 - 1e-07)
RHO_CLIP_HI = N_RHO - 1 - 1e-04

NC = 2
NS = 16
NW = NC * NS
L = 16

NA_PAD = 100352
AT_W = NA_PAD // NW
AT_ROWS = AT_W // L
AT_SC = NA_PAD // NS

PAIRS_T = N_PAIRS // NS
CHUNK_A = 800
VECS_A = CHUNK_A // L
NCHUNK_A = PAIRS_T // CHUNK_A

PAIRS_W = N_PAIRS // NW
CHUNK_C = 800
VECS_C = CHUNK_C // L
NCHUNK_C = PAIRS_W // CHUNK_C

DENS_N = E_TYPES * N_R
PAIR_N = E_TYPES * E_TYPES * N_R
OUT_LEN = N_ATOMS + N_PAIRS
ECHUNKS = N_ATOMS // CHUNK_C

f32 = jnp.float32
i32 = jnp.int32


def _rbin(rr):
    rc = jnp.minimum(jnp.maximum(rr, 0.0), RMAX_C)
    idxf = rc * INV_DR
    idx = idxf.astype(i32)
    frac = idxf - idx.astype(f32)
    return idx, frac


def _vloop(n, body):
    def wrap(i, carry):
        body(i)
        return carry
    lax.fori_loop(0, n, wrap, 0)


def _ploop(n, body, unroll=4):
    plsc.parallel_loop(0, n, unroll=unroll)(body)


def _zero_fill(ref, nwords):
    zeros16 = jnp.zeros((L,), f32)

    def zf(i):
        ref[pl.ds(i * L, L)] = zeros16

    _vloop(nwords // L, zf)


def _pair_pass1_body(r1, s1, d1, tpad, dflat, pflat,
                     rho_out, pe_out, dbi_out, dbj_out,
                     type_s, dens_ts, pair_ts, acc,
                     r_b0, s_b0, d_b0, r_b1, s_b1, d_b1,
                     ti_b, tj_b, i0_b, i1_b, v0_b, v1_b,
                     sem_in, sem_t, sem_v, sem_o):
    c = lax.axis_index("c")
    s = lax.axis_index("s")

    sl = pl.ds(s * AT_SC, AT_SC)
    pltpu.sync_copy(tpad.at[sl], type_s.at[sl])
    dsl = pl.ds(s * (DENS_N // NS), DENS_N // NS)
    pltpu.sync_copy(dflat.at[dsl], dens_ts.at[dsl])
    psl = pl.ds(s * (PAIR_N // NS), PAIR_N // NS)
    pltpu.sync_copy(pflat.at[psl], pair_ts.at[psl])
    _zero_fill(acc, NA_PAD)
    plsc.subcore_barrier()

    bufs = ((r_b0, s_b0, d_b0), (r_b1, s_b1, d_b1))

    def issue_stage(k, bset):
        base = s * PAIRS_T + k * CHUNK_A
        pltpu.async_copy(r1.at[pl.ds(base, CHUNK_A)], bset[0], sem_in)
        pltpu.async_copy(s1.at[pl.ds(base, CHUNK_A)], bset[1], sem_in)
        pltpu.async_copy(d1.at[pl.ds(base, CHUNK_A)], bset[2], sem_in)

    def wait_stage(bset):
        pltpu.make_async_copy(r1.at[pl.ds(0, CHUNK_A)], bset[0],
                              sem_in).wait()
        pltpu.make_async_copy(s1.at[pl.ds(0, CHUNK_A)], bset[1],
                              sem_in).wait()
        pltpu.make_async_copy(d1.at[pl.ds(0, CHUNK_A)], bset[2],
                              sem_in).wait()

    @pl.when(c == 0)
    def _():
        def issue_types(bset):
            pltpu.async_copy(type_s.at[bset[1]], ti_b, sem_t)
            pltpu.async_copy(type_s.at[bset[2]], tj_b, sem_t)

        def wait_types(bset):
            pltpu.make_async_copy(type_s.at[bset[1]], ti_b, sem_t).wait()
            pltpu.make_async_copy(type_s.at[bset[2]], tj_b, sem_t).wait()

        def subbody(k, cur, nxt):
            next_k = k + 1

            @pl.when(next_k < NCHUNK_A)
            def _():
                issue_stage(next_k, nxt)

            wait_types(cur)

            def idxrow(i):
                rsl = pl.ds(i * L, L)
                idx, _ = _rbin(cur[0][rsl])
                dbi = ti_b[rsl] * N_R
                dbj = tj_b[rsl] * N_R
                fi = dbi * E_TYPES + dbj + idx
                i0_b[rsl] = fi
                i1_b[rsl] = fi + 1
                ti_b[rsl] = dbi
                tj_b[rsl] = dbj

            _ploop(VECS_A, idxrow)
            base = s * PAIRS_T + k * CHUNK_A
            pltpu.async_copy(pair_ts.at[i0_b], v0_b, sem_v)
            pltpu.async_copy(pair_ts.at[i1_b], v1_b, sem_v)
            pltpu.async_copy(ti_b, dbi_out.at[pl.ds(base, CHUNK_A)], sem_o)
            pltpu.async_copy(tj_b, dbj_out.at[pl.ds(base, CHUNK_A)], sem_o)

            def wait_o():
                pltpu.make_async_copy(
                    ti_b, dbi_out.at[pl.ds(0, CHUNK_A)], sem_o).wait()
                pltpu.make_async_copy(
                    tj_b, dbj_out.at[pl.ds(0, CHUNK_A)], sem_o).wait()

            @pl.when(next_k < NCHUNK_A)
            def _():
                wait_stage(nxt)
                wait_o()
                issue_types(nxt)

            @pl.when(next_k >= NCHUNK_A)
            def _():
                wait_o()

            pltpu.make_async_copy(pair_ts.at[i0_b], v0_b, sem_v).wait()
            pltpu.make_async_copy(pair_ts.at[i1_b], v1_b, sem_v).wait()

            def accrow(i):
                rsl = pl.ds(i * L, L)
                _, frac = _rbin(cur[0][rsl])
                v0 = v0_b[rsl]
                phi = v0 + frac * (v1_b[rsl] - v0)
                plsc.addupdate_scatter(acc, [cur[1][rsl]], 0.5 * phi)

            _ploop(VECS_A, accrow)

        issue_stage(0, bufs[0])
        wait_stage(bufs[0])
        issue_types(bufs[0])

        def pair_iter(kk):
            subbody(2 * kk, bufs[0], bufs[1])
            subbody(2 * kk + 1, bufs[1], bufs[0])

        _vloop(NCHUNK_A // 2, pair_iter)
        pltpu.sync_copy(acc, pe_out.at[pl.ds(s * NA_PAD, NA_PAD)])

    @pl.when(c == 1)
    def _():
        def issue_types(bset):
            pltpu.async_copy(type_s.at[bset[2]], tj_b, sem_t)

        def wait_types(bset):
            pltpu.make_async_copy(type_s.at[bset[2]], tj_b, sem_t).wait()

        def subbody(k, cur, nxt):
            next_k = k + 1

            @pl.when(next_k < NCHUNK_A)
            def _():
                issue_stage(next_k, nxt)

            wait_types(cur)

            def idxrow(i):
                rsl = pl.ds(i * L, L)
                idx, _ = _rbin(cur[0][rsl])
                fi = tj_b[rsl] * N_R + idx
                i0_b[rsl] = fi
                i1_b[rsl] = fi + 1

            _ploop(VECS_A, idxrow)
            pltpu.async_copy(dens_ts.at[i0_b], v0_b, sem_v)
            pltpu.async_copy(dens_ts.at[i1_b], v1_b, sem_v)

            @pl.when(next_k < NCHUNK_A)
            def _():
                wait_stage(nxt)
                issue_types(nxt)

            pltpu.make_async_copy(dens_ts.at[i0_b], v0_b, sem_v).wait()
            pltpu.make_async_copy(dens_ts.at[i1_b], v1_b, sem_v).wait()

            def accrow(i):
                rsl = pl.ds(i * L, L)
                _, frac = _rbin(cur[0][rsl])
                v0 = v0_b[rsl]
                dens = v0 + frac * (v1_b[rsl] - v0)
                plsc.addupdate_scatter(acc, [cur[1][rsl]], dens)

            _ploop(VECS_A, accrow)

        issue_stage(0, bufs[0])
        wait_stage(bufs[0])
        issue_types(bufs[0])

        def pair_iter(kk):
            subbody(2 * kk, bufs[0], bufs[1])
            subbody(2 * kk + 1, bufs[1], bufs[0])

        _vloop(NCHUNK_A // 2, pair_iter)
        pltpu.sync_copy(acc, rho_out.at[pl.ds(s * NA_PAD, NA_PAD)])


def _atom_pass_body(rho_part, pe_part, tpad, eflat, epflat, rmin16, idr16,
                    en_out, fp_out,
                    embed_t, embedp_t, rmin_t, idr_t,
                    rho_b, pe_b, tmp_b, tmp2_b, tb, en_b, fp_b, sem_r):
    c = lax.axis_index("c")
    s = lax.axis_index("s")
    w = c * NS + s
    base = pl.ds(w * AT_W, AT_W)

    pltpu.sync_copy(eflat, embed_t)
    pltpu.sync_copy(epflat, embedp_t)
    pltpu.sync_copy(rmin16, rmin_t)
    pltpu.sync_copy(idr16, idr_t)
    pltpu.sync_copy(tpad.at[base], tb)

    pltpu.sync_copy(rho_part.at[pl.ds(w * AT_W, AT_W)], rho_b)
    pltpu.sync_copy(pe_part.at[pl.ds(w * AT_W, AT_W)], pe_b)

    def red(p):
        cp_r = pltpu.async_copy(
            rho_part.at[pl.ds(p * NA_PAD + w * AT_W, AT_W)], tmp_b, sem_r)
        cp_p = pltpu.async_copy(
            pe_part.at[pl.ds(p * NA_PAD + w * AT_W, AT_W)], tmp2_b, sem_r)
        cp_r.wait()

        def addrow_r(j):
            jsl = pl.ds(j * L, L)
            rho_b[jsl] = rho_b[jsl] + tmp_b[jsl]

        _vloop(AT_ROWS, addrow_r)
        cp_p.wait()

        def addrow_p(j):
            jsl = pl.ds(j * L, L)
            pe_b[jsl] = pe_b[jsl] + tmp2_b[jsl]

        _vloop(AT_ROWS, addrow_p)

    def redwrap(p, carry):
        red(p + 1)
        return carry

    lax.fori_loop(0, NS - 1, redwrap, 0)

    def row(j):
        jsl = pl.ds(j * L, L)
        t = tb[jsl]
        rho = rho_b[jsl]
        rm = plsc.load_gather(rmin_t, [t])
        iv = plsc.load_gather(idr_t, [t])
        idxf = jnp.minimum(jnp.maximum((rho - rm) * iv, 0.0), RHO_CLIP_HI)
        idx = idxf.astype(i32)
        frac = idxf - idx.astype(f32)
        eb = t * N_RHO + idx
        F0 = plsc.load_gather(embed_t, [eb])
        F1 = plsc.load_gather(embed_t, [eb + 1])
        G0 = plsc.load_gather(embedp_t, [eb])
        G1 = plsc.load_gather(embedp_t, [eb + 1])
        en_b[jsl] = F0 + frac * (F1 - F0) + pe_b[jsl]
        fp_b[jsl] = G0 + frac * (G1 - G0)

    _vloop(AT_ROWS, row)
    pltpu.sync_copy(en_b, en_out.at[base])
    pltpu.sync_copy(fp_b, fp_out.at[base])


def _pair_pass2_body(r1, s1, d1, dbi, dbj, fp_pad, dpflat, ppflat, en_pad,
                     out1,
                     fp_s, densp_t, pairp_t,
                     r_b0, s_b0, d_b0, bi_b0, bj_b0, fs_b0, fd_b0,
                     r_b1, s_b1, d_b1, bi_b1, bj_b1, fs_b1, fd_b1,
                     f_b0, f_b1,
                     sem_in, sem_st, sem_out):
    c = lax.axis_index("c")
    s = lax.axis_index("s")
    w = c * NS + s

    sl = pl.ds(s * AT_SC, AT_SC)
    pltpu.sync_copy(fp_pad.at[sl], fp_s.at[sl])
    pltpu.sync_copy(dpflat, densp_t)
    pltpu.sync_copy(ppflat, pairp_t)

    nch = ECHUNKS // NW + jnp.where(w < ECHUNKS % NW, 1, 0)

    def ecopy(k):
        ebase = (w + k * NW) * CHUNK_C
        pltpu.sync_copy(en_pad.at[pl.ds(ebase, CHUNK_C)], f_b0)
        pltpu.sync_copy(f_b0, out1.at[pl.ds(ebase, CHUNK_C)])

    _vloop(nch, ecopy)
    plsc.subcore_barrier()

    bufs = ((r_b0, s_b0, d_b0, bi_b0, bj_b0, fs_b0, fd_b0, f_b0),
            (r_b1, s_b1, d_b1, bi_b1, bj_b1, fs_b1, fd_b1, f_b1))

    def issue_stage(k, bset):
        base = w * PAIRS_W + k * CHUNK_C
        pltpu.async_copy(r1.at[pl.ds(base, CHUNK_C)], bset[0], sem_in)
        pltpu.async_copy(s1.at[pl.ds(base, CHUNK_C)], bset[1], sem_in)
        pltpu.async_copy(d1.at[pl.ds(base, CHUNK_C)], bset[2], sem_in)
        pltpu.async_copy(dbi.at[pl.ds(base, CHUNK_C)], bset[3], sem_in)
        pltpu.async_copy(dbj.at[pl.ds(base, CHUNK_C)], bset[4], sem_in)

    def wait_stage(bset):
        for ref, hb in ((bset[0], r1), (bset[1], s1), (bset[2], d1),
                        (bset[3], dbi), (bset[4], dbj)):
            pltpu.make_async_copy(hb.at[pl.ds(0, CHUNK_C)], ref,
                                  sem_in).wait()

    def issue_fp(bset):
        pltpu.async_copy(fp_s.at[bset[1]], bset[5], sem_st)
        pltpu.async_copy(fp_s.at[bset[2]], bset[6], sem_st)

    def wait_fp(bset):
        pltpu.make_async_copy(fp_s.at[bset[1]], bset[5], sem_st).wait()
        pltpu.make_async_copy(fp_s.at[bset[2]], bset[6], sem_st).wait()

    def subbody(k, cur, nxt):
        next_k = k + 1

        @pl.when(next_k < NCHUNK_C)
        def _():
            issue_stage(next_k, nxt)

        @pl.when(k >= 2)
        def _():
            pltpu.make_async_copy(
                cur[7], out1.at[pl.ds(N_ATOMS, CHUNK_C)], sem_out).wait()

        wait_fp(cur)

        @pl.when(next_k < NCHUNK_C)
        def _():
            wait_stage(nxt)
            issue_fp(nxt)

        def row(i):
            rsl = pl.ds(i * L, L)
            idx, frac = _rbin(cur[0][rsl])
            bi = cur[3][rsl]
            bj = cur[4][rsl]
            pb = bi * E_TYPES + bj + idx
            p0 = plsc.load_gather(pairp_t, [pb])
            p1 = plsc.load_gather(pairp_t, [pb + 1])
            phip = p0 + frac * (p1 - p0)
            j0 = plsc.load_gather(densp_t, [bj + idx])
            j1 = plsc.load_gather(densp_t, [bj + idx + 1])
            rhop_j = j0 + frac * (j1 - j0)
            q0 = plsc.load_gather(densp_t, [bi + idx])
            q1 = plsc.load_gather(densp_t, [bi + idx + 1])
            rhop_i = q0 + frac * (q1 - q0)
            cur[7][rsl] = phip + cur[5][rsl] * rhop_j + cur[6][rsl] * rhop_i

        _ploop(VECS_C, row)
        base = w * PAIRS_W + k * CHUNK_C
        pltpu.async_copy(cur[7], out1.at[pl.ds(N_ATOMS + base, CHUNK_C)],
                         sem_out)

    issue_stage(0, bufs[0])
    wait_stage(bufs[0])
    issue_fp(bufs[0])

    def pair_iter(kk):
        subbody(2 * kk, bufs[0], bufs[1])
        subbody(2 * kk + 1, bufs[1], bufs[0])

    _vloop(NCHUNK_C // 2, pair_iter)
    if NCHUNK_C % 2:
        subbody(NCHUNK_C - 1, bufs[0], bufs[1])
    pltpu.make_async_copy(f_b0, out1.at[pl.ds(N_ATOMS, CHUNK_C)],
                          sem_out).wait()
    pltpu.make_async_copy(f_b1, out1.at[pl.ds(N_ATOMS, CHUNK_C)],
                          sem_out).wait()


@functools.cache
def _build(interpret=False):
    def mesh():
        return plsc.VectorSubcoreMesh(core_axis_name="c",
                                      subcore_axis_name="s")

    params = pltpu.CompilerParams(needs_layout_passes=False)

    pass1 = pl.kernel(
        _pair_pass1_body,
        out_type=(
            jax.ShapeDtypeStruct((NS * NA_PAD,), f32),
            jax.ShapeDtypeStruct((NS * NA_PAD,), f32),
            jax.ShapeDtypeStruct((N_PAIRS,), i32),
            jax.ShapeDtypeStruct((N_PAIRS,), i32),
        ),
        mesh=mesh(),
        interpret=interpret,
        compiler_params=params,
        scratch_types=[
            pltpu.VMEM_SHARED((NA_PAD,), i32),
            pltpu.VMEM_SHARED((DENS_N,), f32),
            pltpu.VMEM_SHARED((PAIR_N,), f32),
            pltpu.VMEM((NA_PAD,), f32),
            pltpu.VMEM((CHUNK_A,), f32),
            pltpu.VMEM((CHUNK_A,), i32),
            pltpu.VMEM((CHUNK_A,), i32),
            pltpu.VMEM((CHUNK_A,), f32),
            pltpu.VMEM((CHUNK_A,), i32),
            pltpu.VMEM((CHUNK_A,), i32),
            pltpu.VMEM((CHUNK_A,), i32),
            pltpu.VMEM((CHUNK_A,), i32),
            pltpu.VMEM((CHUNK_A,), i32),
            pltpu.VMEM((CHUNK_A,), i32),
            pltpu.VMEM((CHUNK_A,), f32),
            pltpu.VMEM((CHUNK_A,), f32),
            pltpu.SemaphoreType.DMA,
            pltpu.SemaphoreType.DMA,
            pltpu.SemaphoreType.DMA,
            pltpu.SemaphoreType.DMA,
        ],
    )

    pass_b = pl.kernel(
        _atom_pass_body,
        out_type=(
            jax.ShapeDtypeStruct((NA_PAD,), f32),
            jax.ShapeDtypeStruct((NA_PAD,), f32),
        ),
        mesh=mesh(),
        interpret=interpret,
        compiler_params=params,
        scratch_types=[
            pltpu.VMEM((E_TYPES * N_RHO,), f32),
            pltpu.VMEM((E_TYPES * N_RHO,), f32),
            pltpu.VMEM((L,), f32),
            pltpu.VMEM((L,), f32),
            pltpu.VMEM((AT_W,), f32),
            pltpu.VMEM((AT_W,), f32),
            pltpu.VMEM((AT_W,), f32),
            pltpu.VMEM((AT_W,), f32),
            pltpu.VMEM((AT_W,), i32),
            pltpu.VMEM((AT_W,), f32),
            pltpu.VMEM((AT_W,), f32),
            pltpu.SemaphoreType.DMA,
        ],
    )

    pass2 = pl.kernel(
        _pair_pass2_body,
        out_type=jax.ShapeDtypeStruct((OUT_LEN,), f32),
        mesh=mesh(),
        interpret=interpret,
        compiler_params=params,
        scratch_types=[
            pltpu.VMEM_SHARED((NA_PAD,), f32),
            pltpu.VMEM((DENS_N,), f32),
            pltpu.VMEM((PAIR_N,), f32),
            pltpu.VMEM((CHUNK_C,), f32),
            pltpu.VMEM((CHUNK_C,), i32),
            pltpu.VMEM((CHUNK_C,), i32),
            pltpu.VMEM((CHUNK_C,), i32),
            pltpu.VMEM((CHUNK_C,), i32),
            pltpu.VMEM((CHUNK_C,), f32),
            pltpu.VMEM((CHUNK_C,), f32),
            pltpu.VMEM((CHUNK_C,), f32),
            pltpu.VMEM((CHUNK_C,), i32),
            pltpu.VMEM((CHUNK_C,), i32),
            pltpu.VMEM((CHUNK_C,), i32),
            pltpu.VMEM((CHUNK_C,), i32),
            pltpu.VMEM((CHUNK_C,), f32),
            pltpu.VMEM((CHUNK_C,), f32),
            pltpu.VMEM((CHUNK_C,), f32),
            pltpu.VMEM((CHUNK_C,), f32),
            pltpu.SemaphoreType.DMA,
            pltpu.SemaphoreType.DMA,
            pltpu.SemaphoreType.DMA,
        ],
    )
    return pass1, pass_b, pass2


def _run(r, edge_index, atom_type_indices, density_table,
         density_deriv_table, pair_table, pair_deriv_table,
         embed_table, embed_deriv_table, embed_rho_min, embed_inv_drho,
         interpret=False):
    pass1, pass_b, pass2 = _build(interpret)
    src1 = edge_index[0]
    dst1 = edge_index[1]
    tpad = jnp.pad(atom_type_indices, (0, NA_PAD - N_ATOMS))
    rmin16 = jnp.pad(embed_rho_min, (0, L - E_TYPES))
    idr16 = jnp.pad(embed_inv_drho, (0, L - E_TYPES))

    rho_part, pe_part, dbi, dbj = pass1(
        r, src1, dst1, tpad,
        density_table.reshape(-1), pair_table.reshape(-1))
    en_pad, fp_pad = pass_b(
        rho_part, pe_part, tpad,
        embed_table.reshape(-1), embed_deriv_table.reshape(-1),
        rmin16, idr16)
    return pass2(
        r, src1, dst1, dbi, dbj, fp_pad,
        density_deriv_table.reshape(-1), pair_deriv_table.reshape(-1),
        en_pad)


def kernel(r, edge_index, atom_type_indices, density_table,
           density_deriv_table, pair_table, pair_deriv_table,
           embed_table, embed_deriv_table, embed_rho_min, embed_inv_drho):
    return _run(r, edge_index, atom_type_indices, density_table,
                density_deriv_table, pair_table, pair_deriv_table,
                embed_table, embed_deriv_table, embed_rho_min,
                embed_inv_drho)


# speedup vs baseline: 2.8939x; 1.0391x over previous
import functools

import jax
import jax.numpy as jnp
from jax import lax
from jax.experimental import pallas as pl
from jax.experimental.pallas import tpu as pltpu
from jax.experimental.pallas import tpu_sc as plsc

N_ATOMS = 100000
N_PAIRS = 3200000
E_TYPES = 3
N_R = 8192
N_RHO = 4096
R_MAX = 6.0
INV_DR = (N_R - 1) / R_MAX
RMAX_C = R_MAX * (1.0 - 1e-07)
RHO_CLIP_HI = N_RHO - 1 - 1e-04

NC = 2
NS = 16
NW = NC * NS
L = 16

NA_PAD = 100352
AT_W = NA_PAD // NW
AT_ROWS = AT_W // L
AT_SC = NA_PAD // NS

PAIRS_T = N_PAIRS // NS
CHUNK_A = 800
VECS_A = CHUNK_A // L
NCHUNK_A = PAIRS_T // CHUNK_A

PAIRS_W = N_PAIRS // NW
CHUNK_C = 800
VECS_C = CHUNK_C // L
NCHUNK_C = PAIRS_W // CHUNK_C

DENS_N = E_TYPES * N_R
PAIR_N = E_TYPES * E_TYPES * N_R
OUT_LEN = N_ATOMS + N_PAIRS
ECHUNKS = N_ATOMS // CHUNK_C

f32 = jnp.float32
i32 = jnp.int32


def _rbin(rr):
    rc = jnp.minimum(jnp.maximum(rr, 0.0), RMAX_C)
    idxf = rc * INV_DR
    idx = idxf.astype(i32)
    frac = idxf - idx.astype(f32)
    return idx, frac


def _vloop(n, body):
    def wrap(i, carry):
        body(i)
        return carry
    lax.fori_loop(0, n, wrap, 0)


def _ploop(n, body, unroll=4):
    plsc.parallel_loop(0, n, unroll=unroll)(body)


def _zero_fill(ref, nwords):
    zeros16 = jnp.zeros((L,), f32)

    def zf(i):
        ref[pl.ds(i * L, L)] = zeros16

    _vloop(nwords // L, zf)


def _pair_pass1_body(r1, s1, d1, tpad, dflat, pflat,
                     rho_out, pe_out, dbi_out, dbj_out,
                     type_s, dens_ts, pair_ts, acc,
                     r_b0, s_b0, d_b0, r_b1, s_b1, d_b1,
                     ti_b, tj_b, i0_b, i1_b, v0_b, v1_b,
                     sem_in, sem_t, sem_v, sem_o):
    c = lax.axis_index("c")
    s = lax.axis_index("s")

    sl = pl.ds(s * AT_SC, AT_SC)
    pltpu.sync_copy(tpad.at[sl], type_s.at[sl])
    dsl = pl.ds(s * (DENS_N // NS), DENS_N // NS)
    pltpu.sync_copy(dflat.at[dsl], dens_ts.at[dsl])
    psl = pl.ds(s * (PAIR_N // NS), PAIR_N // NS)
    pltpu.sync_copy(pflat.at[psl], pair_ts.at[psl])
    _zero_fill(acc, NA_PAD)
    plsc.subcore_barrier()

    bufs = ((r_b0, s_b0, d_b0), (r_b1, s_b1, d_b1))

    def issue_stage(k, bset):
        base = s * PAIRS_T + k * CHUNK_A
        pltpu.async_copy(r1.at[pl.ds(base, CHUNK_A)], bset[0], sem_in)
        pltpu.async_copy(s1.at[pl.ds(base, CHUNK_A)], bset[1], sem_in)
        pltpu.async_copy(d1.at[pl.ds(base, CHUNK_A)], bset[2], sem_in)

    def wait_stage(bset):
        pltpu.make_async_copy(r1.at[pl.ds(0, CHUNK_A)], bset[0],
                              sem_in).wait()
        pltpu.make_async_copy(s1.at[pl.ds(0, CHUNK_A)], bset[1],
                              sem_in).wait()
        pltpu.make_async_copy(d1.at[pl.ds(0, CHUNK_A)], bset[2],
                              sem_in).wait()

    @pl.when(c == 0)
    def _():
        def issue_types(bset):
            pltpu.async_copy(type_s.at[bset[1]], ti_b, sem_t)
            pltpu.async_copy(type_s.at[bset[2]], tj_b, sem_t)

        def wait_types(bset):
            pltpu.make_async_copy(type_s.at[bset[1]], ti_b, sem_t).wait()
            pltpu.make_async_copy(type_s.at[bset[2]], tj_b, sem_t).wait()

        def subbody(k, cur, nxt):
            next_k = k + 1

            @pl.when(next_k < NCHUNK_A)
            def _():
                issue_stage(next_k, nxt)

            wait_types(cur)

            def idxrow(i):
                rsl = pl.ds(i * L, L)
                idx, _ = _rbin(cur[0][rsl])
                dbi = ti_b[rsl] * N_R
                dbj = tj_b[rsl] * N_R
                fi = dbi * E_TYPES + dbj + idx
                i0_b[rsl] = fi
                i1_b[rsl] = fi + 1
                ti_b[rsl] = dbi
                tj_b[rsl] = dbj

            _ploop(VECS_A, idxrow)
            base = s * PAIRS_T + k * CHUNK_A
            pltpu.async_copy(pair_ts.at[i0_b], v0_b, sem_v)
            pltpu.async_copy(pair_ts.at[i1_b], v1_b, sem_v)
            pltpu.async_copy(ti_b, dbi_out.at[pl.ds(base, CHUNK_A)], sem_o)
            pltpu.async_copy(tj_b, dbj_out.at[pl.ds(base, CHUNK_A)], sem_o)

            def wait_o():
                pltpu.make_async_copy(
                    ti_b, dbi_out.at[pl.ds(0, CHUNK_A)], sem_o).wait()
                pltpu.make_async_copy(
                    tj_b, dbj_out.at[pl.ds(0, CHUNK_A)], sem_o).wait()

            @pl.when(next_k < NCHUNK_A)
            def _():
                wait_stage(nxt)
                wait_o()
                issue_types(nxt)

            @pl.when(next_k >= NCHUNK_A)
            def _():
                wait_o()

            pltpu.make_async_copy(pair_ts.at[i0_b], v0_b, sem_v).wait()
            pltpu.make_async_copy(pair_ts.at[i1_b], v1_b, sem_v).wait()

            def accrow(i):
                rsl = pl.ds(i * L, L)
                _, frac = _rbin(cur[0][rsl])
                v0 = v0_b[rsl]
                phi = v0 + frac * (v1_b[rsl] - v0)
                plsc.addupdate_scatter(acc, [cur[1][rsl]], 0.5 * phi)

            _ploop(VECS_A, accrow)

        issue_stage(0, bufs[0])
        wait_stage(bufs[0])
        issue_types(bufs[0])

        def pair_iter(kk):
            subbody(2 * kk, bufs[0], bufs[1])
            subbody(2 * kk + 1, bufs[1], bufs[0])

        _vloop(NCHUNK_A // 2, pair_iter)
        pltpu.sync_copy(acc, pe_out.at[pl.ds(s * NA_PAD, NA_PAD)])

    @pl.when(c == 1)
    def _():
        def issue_types(bset):
            pltpu.async_copy(type_s.at[bset[2]], tj_b, sem_t)

        def wait_types(bset):
            pltpu.make_async_copy(type_s.at[bset[2]], tj_b, sem_t).wait()

        def subbody(k, cur, nxt):
            next_k = k + 1

            @pl.when(next_k < NCHUNK_A)
            def _():
                issue_stage(next_k, nxt)

            wait_types(cur)

            def idxrow(i):
                rsl = pl.ds(i * L, L)
                idx, _ = _rbin(cur[0][rsl])
                fi = tj_b[rsl] * N_R + idx
                i0_b[rsl] = fi
                i1_b[rsl] = fi + 1

            _ploop(VECS_A, idxrow)
            pltpu.async_copy(dens_ts.at[i0_b], v0_b, sem_v)
            pltpu.async_copy(dens_ts.at[i1_b], v1_b, sem_v)

            @pl.when(next_k < NCHUNK_A)
            def _():
                wait_stage(nxt)
                issue_types(nxt)

            pltpu.make_async_copy(dens_ts.at[i0_b], v0_b, sem_v).wait()
            pltpu.make_async_copy(dens_ts.at[i1_b], v1_b, sem_v).wait()

            def accrow(i):
                rsl = pl.ds(i * L, L)
                _, frac = _rbin(cur[0][rsl])
                v0 = v0_b[rsl]
                dens = v0 + frac * (v1_b[rsl] - v0)
                plsc.addupdate_scatter(acc, [cur[1][rsl]], dens)

            _ploop(VECS_A, accrow)

        issue_stage(0, bufs[0])
        wait_stage(bufs[0])
        issue_types(bufs[0])

        def pair_iter(kk):
            subbody(2 * kk, bufs[0], bufs[1])
            subbody(2 * kk + 1, bufs[1], bufs[0])

        _vloop(NCHUNK_A // 2, pair_iter)
        pltpu.sync_copy(acc, rho_out.at[pl.ds(s * NA_PAD, NA_PAD)])


def _atom_pass_body(rho_part, pe_part, tpad, eflat, epflat, rmin16, idr16,
                    en_out, fp_out,
                    embed_t, embedp_t, rmin_t, idr_t,
                    rho_b, pe_b, tmp_b, tmp2_b, tb, en_b, fp_b, sem_r):
    c = lax.axis_index("c")
    s = lax.axis_index("s")
    w = c * NS + s
    base = pl.ds(w * AT_W, AT_W)

    pltpu.sync_copy(eflat, embed_t)
    pltpu.sync_copy(epflat, embedp_t)
    pltpu.sync_copy(rmin16, rmin_t)
    pltpu.sync_copy(idr16, idr_t)
    pltpu.sync_copy(tpad.at[base], tb)

    pltpu.sync_copy(rho_part.at[pl.ds(w * AT_W, AT_W)], rho_b)
    pltpu.sync_copy(pe_part.at[pl.ds(w * AT_W, AT_W)], pe_b)

    def red(p):
        cp_r = pltpu.async_copy(
            rho_part.at[pl.ds(p * NA_PAD + w * AT_W, AT_W)], tmp_b, sem_r)
        cp_p = pltpu.async_copy(
            pe_part.at[pl.ds(p * NA_PAD + w * AT_W, AT_W)], tmp2_b, sem_r)
        cp_r.wait()

        def addrow_r(j):
            jsl = pl.ds(j * L, L)
            rho_b[jsl] = rho_b[jsl] + tmp_b[jsl]

        _ploop(AT_ROWS, addrow_r)
        cp_p.wait()

        def addrow_p(j):
            jsl = pl.ds(j * L, L)
            pe_b[jsl] = pe_b[jsl] + tmp2_b[jsl]

        _ploop(AT_ROWS, addrow_p)

    def redwrap(p, carry):
        red(p + 1)
        return carry

    lax.fori_loop(0, NS - 1, redwrap, 0)

    def row(j):
        jsl = pl.ds(j * L, L)
        t = tb[jsl]
        rho = rho_b[jsl]
        rm = plsc.load_gather(rmin_t, [t])
        iv = plsc.load_gather(idr_t, [t])
        idxf = jnp.minimum(jnp.maximum((rho - rm) * iv, 0.0), RHO_CLIP_HI)
        idx = idxf.astype(i32)
        frac = idxf - idx.astype(f32)
        eb = t * N_RHO + idx
        F0 = plsc.load_gather(embed_t, [eb])
        F1 = plsc.load_gather(embed_t, [eb + 1])
        G0 = plsc.load_gather(embedp_t, [eb])
        G1 = plsc.load_gather(embedp_t, [eb + 1])
        en_b[jsl] = F0 + frac * (F1 - F0) + pe_b[jsl]
        fp_b[jsl] = G0 + frac * (G1 - G0)

    _ploop(AT_ROWS, row)
    pltpu.sync_copy(en_b, en_out.at[base])
    pltpu.sync_copy(fp_b, fp_out.at[base])


def _pair_pass2_body(r1, s1, d1, dbi, dbj, fp_pad, dpflat, ppflat, en_pad,
                     out1,
                     fp_s, densp_t, pairp_t,
                     r_b0, s_b0, d_b0, bi_b0, bj_b0, fs_b0, fd_b0,
                     r_b1, s_b1, d_b1, bi_b1, bj_b1, fs_b1, fd_b1,
                     f_b0, f_b1,
                     sem_in, sem_st, sem_out):
    c = lax.axis_index("c")
    s = lax.axis_index("s")
    w = c * NS + s

    sl = pl.ds(s * AT_SC, AT_SC)
    pltpu.sync_copy(fp_pad.at[sl], fp_s.at[sl])
    pltpu.sync_copy(dpflat, densp_t)
    pltpu.sync_copy(ppflat, pairp_t)

    nch = ECHUNKS // NW + jnp.where(w < ECHUNKS % NW, 1, 0)

    def ecopy(k):
        ebase = (w + k * NW) * CHUNK_C
        pltpu.sync_copy(en_pad.at[pl.ds(ebase, CHUNK_C)], f_b0)
        pltpu.sync_copy(f_b0, out1.at[pl.ds(ebase, CHUNK_C)])

    _vloop(nch, ecopy)
    plsc.subcore_barrier()

    bufs = ((r_b0, s_b0, d_b0, bi_b0, bj_b0, fs_b0, fd_b0, f_b0),
            (r_b1, s_b1, d_b1, bi_b1, bj_b1, fs_b1, fd_b1, f_b1))

    def issue_stage(k, bset):
        base = w * PAIRS_W + k * CHUNK_C
        pltpu.async_copy(r1.at[pl.ds(base, CHUNK_C)], bset[0], sem_in)
        pltpu.async_copy(s1.at[pl.ds(base, CHUNK_C)], bset[1], sem_in)
        pltpu.async_copy(d1.at[pl.ds(base, CHUNK_C)], bset[2], sem_in)
        pltpu.async_copy(dbi.at[pl.ds(base, CHUNK_C)], bset[3], sem_in)
        pltpu.async_copy(dbj.at[pl.ds(base, CHUNK_C)], bset[4], sem_in)

    def wait_stage(bset):
        for ref, hb in ((bset[0], r1), (bset[1], s1), (bset[2], d1),
                        (bset[3], dbi), (bset[4], dbj)):
            pltpu.make_async_copy(hb.at[pl.ds(0, CHUNK_C)], ref,
                                  sem_in).wait()

    def issue_fp(bset):
        pltpu.async_copy(fp_s.at[bset[1]], bset[5], sem_st)
        pltpu.async_copy(fp_s.at[bset[2]], bset[6], sem_st)

    def wait_fp(bset):
        pltpu.make_async_copy(fp_s.at[bset[1]], bset[5], sem_st).wait()
        pltpu.make_async_copy(fp_s.at[bset[2]], bset[6], sem_st).wait()

    def subbody(k, cur, nxt):
        next_k = k + 1

        @pl.when(next_k < NCHUNK_C)
        def _():
            issue_stage(next_k, nxt)

        @pl.when(k >= 2)
        def _():
            pltpu.make_async_copy(
                cur[7], out1.at[pl.ds(N_ATOMS, CHUNK_C)], sem_out).wait()

        wait_fp(cur)

        @pl.when(next_k < NCHUNK_C)
        def _():
            wait_stage(nxt)
            issue_fp(nxt)

        def row(i):
            rsl = pl.ds(i * L, L)
            idx, frac = _rbin(cur[0][rsl])
            bi = cur[3][rsl]
            bj = cur[4][rsl]
            pb = bi * E_TYPES + bj + idx
            p0 = plsc.load_gather(pairp_t, [pb])
            p1 = plsc.load_gather(pairp_t, [pb + 1])
            phip = p0 + frac * (p1 - p0)
            j0 = plsc.load_gather(densp_t, [bj + idx])
            j1 = plsc.load_gather(densp_t, [bj + idx + 1])
            rhop_j = j0 + frac * (j1 - j0)
            q0 = plsc.load_gather(densp_t, [bi + idx])
            q1 = plsc.load_gather(densp_t, [bi + idx + 1])
            rhop_i = q0 + frac * (q1 - q0)
            cur[7][rsl] = phip + cur[5][rsl] * rhop_j + cur[6][rsl] * rhop_i

        _ploop(VECS_C, row)
        base = w * PAIRS_W + k * CHUNK_C
        pltpu.async_copy(cur[7], out1.at[pl.ds(N_ATOMS + base, CHUNK_C)],
                         sem_out)

    issue_stage(0, bufs[0])
    wait_stage(bufs[0])
    issue_fp(bufs[0])

    def pair_iter(kk):
        subbody(2 * kk, bufs[0], bufs[1])
        subbody(2 * kk + 1, bufs[1], bufs[0])

    _vloop(NCHUNK_C // 2, pair_iter)
    if NCHUNK_C % 2:
        subbody(NCHUNK_C - 1, bufs[0], bufs[1])
    pltpu.make_async_copy(f_b0, out1.at[pl.ds(N_ATOMS, CHUNK_C)],
                          sem_out).wait()
    pltpu.make_async_copy(f_b1, out1.at[pl.ds(N_ATOMS, CHUNK_C)],
                          sem_out).wait()


@functools.cache
def _build(interpret=False):
    def mesh():
        return plsc.VectorSubcoreMesh(core_axis_name="c",
                                      subcore_axis_name="s")

    params = pltpu.CompilerParams(needs_layout_passes=False)

    pass1 = pl.kernel(
        _pair_pass1_body,
        out_type=(
            jax.ShapeDtypeStruct((NS * NA_PAD,), f32),
            jax.ShapeDtypeStruct((NS * NA_PAD,), f32),
            jax.ShapeDtypeStruct((N_PAIRS,), i32),
            jax.ShapeDtypeStruct((N_PAIRS,), i32),
        ),
        mesh=mesh(),
        interpret=interpret,
        compiler_params=params,
        scratch_types=[
            pltpu.VMEM_SHARED((NA_PAD,), i32),
            pltpu.VMEM_SHARED((DENS_N,), f32),
            pltpu.VMEM_SHARED((PAIR_N,), f32),
            pltpu.VMEM((NA_PAD,), f32),
            pltpu.VMEM((CHUNK_A,), f32),
            pltpu.VMEM((CHUNK_A,), i32),
            pltpu.VMEM((CHUNK_A,), i32),
            pltpu.VMEM((CHUNK_A,), f32),
            pltpu.VMEM((CHUNK_A,), i32),
            pltpu.VMEM((CHUNK_A,), i32),
            pltpu.VMEM((CHUNK_A,), i32),
            pltpu.VMEM((CHUNK_A,), i32),
            pltpu.VMEM((CHUNK_A,), i32),
            pltpu.VMEM((CHUNK_A,), i32),
            pltpu.VMEM((CHUNK_A,), f32),
            pltpu.VMEM((CHUNK_A,), f32),
            pltpu.SemaphoreType.DMA,
            pltpu.SemaphoreType.DMA,
            pltpu.SemaphoreType.DMA,
            pltpu.SemaphoreType.DMA,
        ],
    )

    pass_b = pl.kernel(
        _atom_pass_body,
        out_type=(
            jax.ShapeDtypeStruct((NA_PAD,), f32),
            jax.ShapeDtypeStruct((NA_PAD,), f32),
        ),
        mesh=mesh(),
        interpret=interpret,
        compiler_params=params,
        scratch_types=[
            pltpu.VMEM((E_TYPES * N_RHO,), f32),
            pltpu.VMEM((E_TYPES * N_RHO,), f32),
            pltpu.VMEM((L,), f32),
            pltpu.VMEM((L,), f32),
            pltpu.VMEM((AT_W,), f32),
            pltpu.VMEM((AT_W,), f32),
            pltpu.VMEM((AT_W,), f32),
            pltpu.VMEM((AT_W,), f32),
            pltpu.VMEM((AT_W,), i32),
            pltpu.VMEM((AT_W,), f32),
            pltpu.VMEM((AT_W,), f32),
            pltpu.SemaphoreType.DMA,
        ],
    )

    pass2 = pl.kernel(
        _pair_pass2_body,
        out_type=jax.ShapeDtypeStruct((OUT_LEN,), f32),
        mesh=mesh(),
        interpret=interpret,
        compiler_params=params,
        scratch_types=[
            pltpu.VMEM_SHARED((NA_PAD,), f32),
            pltpu.VMEM((DENS_N,), f32),
            pltpu.VMEM((PAIR_N,), f32),
            pltpu.VMEM((CHUNK_C,), f32),
            pltpu.VMEM((CHUNK_C,), i32),
            pltpu.VMEM((CHUNK_C,), i32),
            pltpu.VMEM((CHUNK_C,), i32),
            pltpu.VMEM((CHUNK_C,), i32),
            pltpu.VMEM((CHUNK_C,), f32),
            pltpu.VMEM((CHUNK_C,), f32),
            pltpu.VMEM((CHUNK_C,), f32),
            pltpu.VMEM((CHUNK_C,), i32),
            pltpu.VMEM((CHUNK_C,), i32),
            pltpu.VMEM((CHUNK_C,), i32),
            pltpu.VMEM((CHUNK_C,), i32),
            pltpu.VMEM((CHUNK_C,), f32),
            pltpu.VMEM((CHUNK_C,), f32),
            pltpu.VMEM((CHUNK_C,), f32),
            pltpu.VMEM((CHUNK_C,), f32),
            pltpu.SemaphoreType.DMA,
            pltpu.SemaphoreType.DMA,
            pltpu.SemaphoreType.DMA,
        ],
    )
    return pass1, pass_b, pass2


def _run(r, edge_index, atom_type_indices, density_table,
         density_deriv_table, pair_table, pair_deriv_table,
         embed_table, embed_deriv_table, embed_rho_min, embed_inv_drho,
         interpret=False):
    pass1, pass_b, pass2 = _build(interpret)
    src1 = edge_index[0]
    dst1 = edge_index[1]
    tpad = jnp.pad(atom_type_indices, (0, NA_PAD - N_ATOMS))
    rmin16 = jnp.pad(embed_rho_min, (0, L - E_TYPES))
    idr16 = jnp.pad(embed_inv_drho, (0, L - E_TYPES))

    rho_part, pe_part, dbi, dbj = pass1(
        r, src1, dst1, tpad,
        density_table.reshape(-1), pair_table.reshape(-1))
    en_pad, fp_pad = pass_b(
        rho_part, pe_part, tpad,
        embed_table.reshape(-1), embed_deriv_table.reshape(-1),
        rmin16, idr16)
    return pass2(
        r, src1, dst1, dbi, dbj, fp_pad,
        density_deriv_table.reshape(-1), pair_deriv_table.reshape(-1),
        en_pad)


def kernel(r, edge_index, atom_type_indices, density_table,
           density_deriv_table, pair_table, pair_deriv_table,
           embed_table, embed_deriv_table, embed_rho_min, embed_inv_drho):
    return _run(r, edge_index, atom_type_indices, density_table,
                density_deriv_table, pair_table, pair_deriv_table,
                embed_table, embed_deriv_table, embed_rho_min,
                embed_inv_drho)


# speedup vs baseline: 2.8972x; 1.0011x over previous
import functools

import jax
import jax.numpy as jnp
from jax import lax
from jax.experimental import pallas as pl
from jax.experimental.pallas import tpu as pltpu
from jax.experimental.pallas import tpu_sc as plsc

N_ATOMS = 100000
N_PAIRS = 3200000
E_TYPES = 3
N_R = 8192
N_RHO = 4096
R_MAX = 6.0
INV_DR = (N_R - 1) / R_MAX
RMAX_C = R_MAX * (1.0 - 1e-07)
RHO_CLIP_HI = N_RHO - 1 - 1e-04

NC = 2
NS = 16
NW = NC * NS
L = 16

NA_PAD = 100352
AT_W = NA_PAD // NW
AT_ROWS = AT_W // L
AT_SC = NA_PAD // NS

PAIRS_T = N_PAIRS // NS
CHUNK_A = 800
VECS_A = CHUNK_A // L
NCHUNK_A = PAIRS_T // CHUNK_A

PAIRS_W = N_PAIRS // NW
CHUNK_C = 800
VECS_C = CHUNK_C // L
NCHUNK_C = PAIRS_W // CHUNK_C

DENS_N = E_TYPES * N_R
PAIR_N = E_TYPES * E_TYPES * N_R
OUT_LEN = N_ATOMS + N_PAIRS
ECHUNKS = N_ATOMS // CHUNK_C

f32 = jnp.float32
i32 = jnp.int32


def _rbin(rr):
    rc = jnp.minimum(jnp.maximum(rr, 0.0), RMAX_C)
    idxf = rc * INV_DR
    idx = idxf.astype(i32)
    frac = idxf - idx.astype(f32)
    return idx, frac


def _vloop(n, body):
    def wrap(i, carry):
        body(i)
        return carry
    lax.fori_loop(0, n, wrap, 0)


def _ploop(n, body, unroll=8):
    plsc.parallel_loop(0, n, unroll=unroll)(body)


def _zero_fill(ref, nwords):
    zeros16 = jnp.zeros((L,), f32)

    def zf(i):
        ref[pl.ds(i * L, L)] = zeros16

    _vloop(nwords // L, zf)


def _pair_pass1_body(r1, s1, d1, tpad, dflat, pflat,
                     rho_out, pe_out, dbi_out, dbj_out,
                     type_s, dens_ts, pair_ts, acc,
                     r_b0, s_b0, d_b0, r_b1, s_b1, d_b1,
                     ti_b, tj_b, i0_b, i1_b, v0_b, v1_b,
                     sem_in, sem_t, sem_v, sem_o):
    c = lax.axis_index("c")
    s = lax.axis_index("s")

    sl = pl.ds(s * AT_SC, AT_SC)
    pltpu.sync_copy(tpad.at[sl], type_s.at[sl])
    dsl = pl.ds(s * (DENS_N // NS), DENS_N // NS)
    pltpu.sync_copy(dflat.at[dsl], dens_ts.at[dsl])
    psl = pl.ds(s * (PAIR_N // NS), PAIR_N // NS)
    pltpu.sync_copy(pflat.at[psl], pair_ts.at[psl])
    _zero_fill(acc, NA_PAD)
    plsc.subcore_barrier()

    bufs = ((r_b0, s_b0, d_b0), (r_b1, s_b1, d_b1))

    def issue_stage(k, bset):
        base = s * PAIRS_T + k * CHUNK_A
        pltpu.async_copy(r1.at[pl.ds(base, CHUNK_A)], bset[0], sem_in)
        pltpu.async_copy(s1.at[pl.ds(base, CHUNK_A)], bset[1], sem_in)
        pltpu.async_copy(d1.at[pl.ds(base, CHUNK_A)], bset[2], sem_in)

    def wait_stage(bset):
        pltpu.make_async_copy(r1.at[pl.ds(0, CHUNK_A)], bset[0],
                              sem_in).wait()
        pltpu.make_async_copy(s1.at[pl.ds(0, CHUNK_A)], bset[1],
                              sem_in).wait()
        pltpu.make_async_copy(d1.at[pl.ds(0, CHUNK_A)], bset[2],
                              sem_in).wait()

    @pl.when(c == 0)
    def _():
        def issue_types(bset):
            pltpu.async_copy(type_s.at[bset[1]], ti_b, sem_t)
            pltpu.async_copy(type_s.at[bset[2]], tj_b, sem_t)

        def wait_types(bset):
            pltpu.make_async_copy(type_s.at[bset[1]], ti_b, sem_t).wait()
            pltpu.make_async_copy(type_s.at[bset[2]], tj_b, sem_t).wait()

        def subbody(k, cur, nxt):
            next_k = k + 1

            @pl.when(next_k < NCHUNK_A)
            def _():
                issue_stage(next_k, nxt)

            wait_types(cur)

            def idxrow(i):
                rsl = pl.ds(i * L, L)
                idx, _ = _rbin(cur[0][rsl])
                dbi = ti_b[rsl] * N_R
                dbj = tj_b[rsl] * N_R
                fi = dbi * E_TYPES + dbj + idx
                i0_b[rsl] = fi
                i1_b[rsl] = fi + 1
                ti_b[rsl] = dbi
                tj_b[rsl] = dbj

            _ploop(VECS_A, idxrow)
            base = s * PAIRS_T + k * CHUNK_A
            pltpu.async_copy(pair_ts.at[i0_b], v0_b, sem_v)
            pltpu.async_copy(pair_ts.at[i1_b], v1_b, sem_v)
            pltpu.async_copy(ti_b, dbi_out.at[pl.ds(base, CHUNK_A)], sem_o)
            pltpu.async_copy(tj_b, dbj_out.at[pl.ds(base, CHUNK_A)], sem_o)

            def wait_o():
                pltpu.make_async_copy(
                    ti_b, dbi_out.at[pl.ds(0, CHUNK_A)], sem_o).wait()
                pltpu.make_async_copy(
                    tj_b, dbj_out.at[pl.ds(0, CHUNK_A)], sem_o).wait()

            @pl.when(next_k < NCHUNK_A)
            def _():
                wait_stage(nxt)
                wait_o()
                issue_types(nxt)

            @pl.when(next_k >= NCHUNK_A)
            def _():
                wait_o()

            pltpu.make_async_copy(pair_ts.at[i0_b], v0_b, sem_v).wait()
            pltpu.make_async_copy(pair_ts.at[i1_b], v1_b, sem_v).wait()

            def accrow(i):
                rsl = pl.ds(i * L, L)
                _, frac = _rbin(cur[0][rsl])
                v0 = v0_b[rsl]
                phi = v0 + frac * (v1_b[rsl] - v0)
                plsc.addupdate_scatter(acc, [cur[1][rsl]], 0.5 * phi)

            _ploop(VECS_A, accrow)

        issue_stage(0, bufs[0])
        wait_stage(bufs[0])
        issue_types(bufs[0])

        def pair_iter(kk):
            subbody(2 * kk, bufs[0], bufs[1])
            subbody(2 * kk + 1, bufs[1], bufs[0])

        _vloop(NCHUNK_A // 2, pair_iter)
        pltpu.sync_copy(acc, pe_out.at[pl.ds(s * NA_PAD, NA_PAD)])

    @pl.when(c == 1)
    def _():
        def issue_types(bset):
            pltpu.async_copy(type_s.at[bset[2]], tj_b, sem_t)

        def wait_types(bset):
            pltpu.make_async_copy(type_s.at[bset[2]], tj_b, sem_t).wait()

        def subbody(k, cur, nxt):
            next_k = k + 1

            @pl.when(next_k < NCHUNK_A)
            def _():
                issue_stage(next_k, nxt)

            wait_types(cur)

            def idxrow(i):
                rsl = pl.ds(i * L, L)
                idx, _ = _rbin(cur[0][rsl])
                fi = tj_b[rsl] * N_R + idx
                i0_b[rsl] = fi
                i1_b[rsl] = fi + 1

            _ploop(VECS_A, idxrow)
            pltpu.async_copy(dens_ts.at[i0_b], v0_b, sem_v)
            pltpu.async_copy(dens_ts.at[i1_b], v1_b, sem_v)

            @pl.when(next_k < NCHUNK_A)
            def _():
                wait_stage(nxt)
                issue_types(nxt)

            pltpu.make_async_copy(dens_ts.at[i0_b], v0_b, sem_v).wait()
            pltpu.make_async_copy(dens_ts.at[i1_b], v1_b, sem_v).wait()

            def accrow(i):
                rsl = pl.ds(i * L, L)
                _, frac = _rbin(cur[0][rsl])
                v0 = v0_b[rsl]
                dens = v0 + frac * (v1_b[rsl] - v0)
                plsc.addupdate_scatter(acc, [cur[1][rsl]], dens)

            _ploop(VECS_A, accrow)

        issue_stage(0, bufs[0])
        wait_stage(bufs[0])
        issue_types(bufs[0])

        def pair_iter(kk):
            subbody(2 * kk, bufs[0], bufs[1])
            subbody(2 * kk + 1, bufs[1], bufs[0])

        _vloop(NCHUNK_A // 2, pair_iter)
        pltpu.sync_copy(acc, rho_out.at[pl.ds(s * NA_PAD, NA_PAD)])


def _atom_pass_body(rho_part, pe_part, tpad, eflat, epflat, rmin16, idr16,
                    en_out, fp_out,
                    embed_t, embedp_t, rmin_t, idr_t,
                    rho_b, pe_b, tmp_b, tmp2_b, tb, en_b, fp_b, sem_r):
    c = lax.axis_index("c")
    s = lax.axis_index("s")
    w = c * NS + s
    base = pl.ds(w * AT_W, AT_W)

    pltpu.sync_copy(eflat, embed_t)
    pltpu.sync_copy(epflat, embedp_t)
    pltpu.sync_copy(rmin16, rmin_t)
    pltpu.sync_copy(idr16, idr_t)
    pltpu.sync_copy(tpad.at[base], tb)

    pltpu.sync_copy(rho_part.at[pl.ds(w * AT_W, AT_W)], rho_b)
    pltpu.sync_copy(pe_part.at[pl.ds(w * AT_W, AT_W)], pe_b)

    def red(p):
        cp_r = pltpu.async_copy(
            rho_part.at[pl.ds(p * NA_PAD + w * AT_W, AT_W)], tmp_b, sem_r)
        cp_p = pltpu.async_copy(
            pe_part.at[pl.ds(p * NA_PAD + w * AT_W, AT_W)], tmp2_b, sem_r)
        cp_r.wait()

        def addrow_r(j):
            jsl = pl.ds(j * L, L)
            rho_b[jsl] = rho_b[jsl] + tmp_b[jsl]

        _ploop(AT_ROWS, addrow_r)
        cp_p.wait()

        def addrow_p(j):
            jsl = pl.ds(j * L, L)
            pe_b[jsl] = pe_b[jsl] + tmp2_b[jsl]

        _ploop(AT_ROWS, addrow_p)

    def redwrap(p, carry):
        red(p + 1)
        return carry

    lax.fori_loop(0, NS - 1, redwrap, 0)

    def row(j):
        jsl = pl.ds(j * L, L)
        t = tb[jsl]
        rho = rho_b[jsl]
        rm = plsc.load_gather(rmin_t, [t])
        iv = plsc.load_gather(idr_t, [t])
        idxf = jnp.minimum(jnp.maximum((rho - rm) * iv, 0.0), RHO_CLIP_HI)
        idx = idxf.astype(i32)
        frac = idxf - idx.astype(f32)
        eb = t * N_RHO + idx
        F0 = plsc.load_gather(embed_t, [eb])
        F1 = plsc.load_gather(embed_t, [eb + 1])
        G0 = plsc.load_gather(embedp_t, [eb])
        G1 = plsc.load_gather(embedp_t, [eb + 1])
        en_b[jsl] = F0 + frac * (F1 - F0) + pe_b[jsl]
        fp_b[jsl] = G0 + frac * (G1 - G0)

    _ploop(AT_ROWS, row)
    pltpu.sync_copy(en_b, en_out.at[base])
    pltpu.sync_copy(fp_b, fp_out.at[base])


def _pair_pass2_body(r1, s1, d1, dbi, dbj, fp_pad, dpflat, ppflat, en_pad,
                     out1,
                     fp_s, densp_t, pairp_t,
                     r_b0, s_b0, d_b0, bi_b0, bj_b0, fs_b0, fd_b0,
                     r_b1, s_b1, d_b1, bi_b1, bj_b1, fs_b1, fd_b1,
                     f_b0, f_b1,
                     sem_in, sem_st, sem_out):
    c = lax.axis_index("c")
    s = lax.axis_index("s")
    w = c * NS + s

    sl = pl.ds(s * AT_SC, AT_SC)
    pltpu.sync_copy(fp_pad.at[sl], fp_s.at[sl])
    pltpu.sync_copy(dpflat, densp_t)
    pltpu.sync_copy(ppflat, pairp_t)

    nch = ECHUNKS // NW + jnp.where(w < ECHUNKS % NW, 1, 0)

    def ecopy(k):
        ebase = (w + k * NW) * CHUNK_C
        pltpu.sync_copy(en_pad.at[pl.ds(ebase, CHUNK_C)], f_b0)
        pltpu.sync_copy(f_b0, out1.at[pl.ds(ebase, CHUNK_C)])

    _vloop(nch, ecopy)
    plsc.subcore_barrier()

    bufs = ((r_b0, s_b0, d_b0, bi_b0, bj_b0, fs_b0, fd_b0, f_b0),
            (r_b1, s_b1, d_b1, bi_b1, bj_b1, fs_b1, fd_b1, f_b1))

    def issue_stage(k, bset):
        base = w * PAIRS_W + k * CHUNK_C
        pltpu.async_copy(r1.at[pl.ds(base, CHUNK_C)], bset[0], sem_in)
        pltpu.async_copy(s1.at[pl.ds(base, CHUNK_C)], bset[1], sem_in)
        pltpu.async_copy(d1.at[pl.ds(base, CHUNK_C)], bset[2], sem_in)
        pltpu.async_copy(dbi.at[pl.ds(base, CHUNK_C)], bset[3], sem_in)
        pltpu.async_copy(dbj.at[pl.ds(base, CHUNK_C)], bset[4], sem_in)

    def wait_stage(bset):
        for ref, hb in ((bset[0], r1), (bset[1], s1), (bset[2], d1),
                        (bset[3], dbi), (bset[4], dbj)):
            pltpu.make_async_copy(hb.at[pl.ds(0, CHUNK_C)], ref,
                                  sem_in).wait()

    def issue_fp(bset):
        pltpu.async_copy(fp_s.at[bset[1]], bset[5], sem_st)
        pltpu.async_copy(fp_s.at[bset[2]], bset[6], sem_st)

    def wait_fp(bset):
        pltpu.make_async_copy(fp_s.at[bset[1]], bset[5], sem_st).wait()
        pltpu.make_async_copy(fp_s.at[bset[2]], bset[6], sem_st).wait()

    def subbody(k, cur, nxt):
        next_k = k + 1

        @pl.when(next_k < NCHUNK_C)
        def _():
            issue_stage(next_k, nxt)

        @pl.when(k >= 2)
        def _():
            pltpu.make_async_copy(
                cur[7], out1.at[pl.ds(N_ATOMS, CHUNK_C)], sem_out).wait()

        wait_fp(cur)

        @pl.when(next_k < NCHUNK_C)
        def _():
            wait_stage(nxt)
            issue_fp(nxt)

        def row(i):
            rsl = pl.ds(i * L, L)
            idx, frac = _rbin(cur[0][rsl])
            bi = cur[3][rsl]
            bj = cur[4][rsl]
            pb = bi * E_TYPES + bj + idx
            p0 = plsc.load_gather(pairp_t, [pb])
            p1 = plsc.load_gather(pairp_t, [pb + 1])
            phip = p0 + frac * (p1 - p0)
            j0 = plsc.load_gather(densp_t, [bj + idx])
            j1 = plsc.load_gather(densp_t, [bj + idx + 1])
            rhop_j = j0 + frac * (j1 - j0)
            q0 = plsc.load_gather(densp_t, [bi + idx])
            q1 = plsc.load_gather(densp_t, [bi + idx + 1])
            rhop_i = q0 + frac * (q1 - q0)
            cur[7][rsl] = phip + cur[5][rsl] * rhop_j + cur[6][rsl] * rhop_i

        _ploop(VECS_C, row)
        base = w * PAIRS_W + k * CHUNK_C
        pltpu.async_copy(cur[7], out1.at[pl.ds(N_ATOMS + base, CHUNK_C)],
                         sem_out)

    issue_stage(0, bufs[0])
    wait_stage(bufs[0])
    issue_fp(bufs[0])

    def pair_iter(kk):
        subbody(2 * kk, bufs[0], bufs[1])
        subbody(2 * kk + 1, bufs[1], bufs[0])

    _vloop(NCHUNK_C // 2, pair_iter)
    if NCHUNK_C % 2:
        subbody(NCHUNK_C - 1, bufs[0], bufs[1])
    pltpu.make_async_copy(f_b0, out1.at[pl.ds(N_ATOMS, CHUNK_C)],
                          sem_out).wait()
    pltpu.make_async_copy(f_b1, out1.at[pl.ds(N_ATOMS, CHUNK_C)],
                          sem_out).wait()


@functools.cache
def _build(interpret=False):
    def mesh():
        return plsc.VectorSubcoreMesh(core_axis_name="c",
                                      subcore_axis_name="s")

    params = pltpu.CompilerParams(needs_layout_passes=False)

    pass1 = pl.kernel(
        _pair_pass1_body,
        out_type=(
            jax.ShapeDtypeStruct((NS * NA_PAD,), f32),
            jax.ShapeDtypeStruct((NS * NA_PAD,), f32),
            jax.ShapeDtypeStruct((N_PAIRS,), i32),
            jax.ShapeDtypeStruct((N_PAIRS,), i32),
        ),
        mesh=mesh(),
        interpret=interpret,
        compiler_params=params,
        scratch_types=[
            pltpu.VMEM_SHARED((NA_PAD,), i32),
            pltpu.VMEM_SHARED((DENS_N,), f32),
            pltpu.VMEM_SHARED((PAIR_N,), f32),
            pltpu.VMEM((NA_PAD,), f32),
            pltpu.VMEM((CHUNK_A,), f32),
            pltpu.VMEM((CHUNK_A,), i32),
            pltpu.VMEM((CHUNK_A,), i32),
            pltpu.VMEM((CHUNK_A,), f32),
            pltpu.VMEM((CHUNK_A,), i32),
            pltpu.VMEM((CHUNK_A,), i32),
            pltpu.VMEM((CHUNK_A,), i32),
            pltpu.VMEM((CHUNK_A,), i32),
            pltpu.VMEM((CHUNK_A,), i32),
            pltpu.VMEM((CHUNK_A,), i32),
            pltpu.VMEM((CHUNK_A,), f32),
            pltpu.VMEM((CHUNK_A,), f32),
            pltpu.SemaphoreType.DMA,
            pltpu.SemaphoreType.DMA,
            pltpu.SemaphoreType.DMA,
            pltpu.SemaphoreType.DMA,
        ],
    )

    pass_b = pl.kernel(
        _atom_pass_body,
        out_type=(
            jax.ShapeDtypeStruct((NA_PAD,), f32),
            jax.ShapeDtypeStruct((NA_PAD,), f32),
        ),
        mesh=mesh(),
        interpret=interpret,
        compiler_params=params,
        scratch_types=[
            pltpu.VMEM((E_TYPES * N_RHO,), f32),
            pltpu.VMEM((E_TYPES * N_RHO,), f32),
            pltpu.VMEM((L,), f32),
            pltpu.VMEM((L,), f32),
            pltpu.VMEM((AT_W,), f32),
            pltpu.VMEM((AT_W,), f32),
            pltpu.VMEM((AT_W,), f32),
            pltpu.VMEM((AT_W,), f32),
            pltpu.VMEM((AT_W,), i32),
            pltpu.VMEM((AT_W,), f32),
            pltpu.VMEM((AT_W,), f32),
            pltpu.SemaphoreType.DMA,
        ],
    )

    pass2 = pl.kernel(
        _pair_pass2_body,
        out_type=jax.ShapeDtypeStruct((OUT_LEN,), f32),
        mesh=mesh(),
        interpret=interpret,
        compiler_params=params,
        scratch_types=[
            pltpu.VMEM_SHARED((NA_PAD,), f32),
            pltpu.VMEM((DENS_N,), f32),
            pltpu.VMEM((PAIR_N,), f32),
            pltpu.VMEM((CHUNK_C,), f32),
            pltpu.VMEM((CHUNK_C,), i32),
            pltpu.VMEM((CHUNK_C,), i32),
            pltpu.VMEM((CHUNK_C,), i32),
            pltpu.VMEM((CHUNK_C,), i32),
            pltpu.VMEM((CHUNK_C,), f32),
            pltpu.VMEM((CHUNK_C,), f32),
            pltpu.VMEM((CHUNK_C,), f32),
            pltpu.VMEM((CHUNK_C,), i32),
            pltpu.VMEM((CHUNK_C,), i32),
            pltpu.VMEM((CHUNK_C,), i32),
            pltpu.VMEM((CHUNK_C,), i32),
            pltpu.VMEM((CHUNK_C,), f32),
            pltpu.VMEM((CHUNK_C,), f32),
            pltpu.VMEM((CHUNK_C,), f32),
            pltpu.VMEM((CHUNK_C,), f32),
            pltpu.SemaphoreType.DMA,
            pltpu.SemaphoreType.DMA,
            pltpu.SemaphoreType.DMA,
        ],
    )
    return pass1, pass_b, pass2


def _run(r, edge_index, atom_type_indices, density_table,
         density_deriv_table, pair_table, pair_deriv_table,
         embed_table, embed_deriv_table, embed_rho_min, embed_inv_drho,
         interpret=False):
    pass1, pass_b, pass2 = _build(interpret)
    src1 = edge_index[0]
    dst1 = edge_index[1]
    tpad = jnp.pad(atom_type_indices, (0, NA_PAD - N_ATOMS))
    rmin16 = jnp.pad(embed_rho_min, (0, L - E_TYPES))
    idr16 = jnp.pad(embed_inv_drho, (0, L - E_TYPES))

    rho_part, pe_part, dbi, dbj = pass1(
        r, src1, dst1, tpad,
        density_table.reshape(-1), pair_table.reshape(-1))
    en_pad, fp_pad = pass_b(
        rho_part, pe_part, tpad,
        embed_table.reshape(-1), embed_deriv_table.reshape(-1),
        rmin16, idr16)
    return pass2(
        r, src1, dst1, dbi, dbj, fp_pad,
        density_deriv_table.reshape(-1), pair_deriv_table.reshape(-1),
        en_pad)


def kernel(r, edge_index, atom_type_indices, density_table,
           density_deriv_table, pair_table, pair_deriv_table,
           embed_table, embed_deriv_table, embed_rho_min, embed_inv_drho):
    return _run(r, edge_index, atom_type_indices, density_table,
                density_deriv_table, pair_table, pair_deriv_table,
                embed_table, embed_deriv_table, embed_rho_min,
                embed_inv_drho)
